# baseline (speedup 1.0000x reference)
import functools

import jax
import jax.numpy as jnp
from jax import lax
from jax.experimental import pallas as pl
from jax.experimental.pallas import tpu as pltpu
from jax.experimental.pallas import tpu_sc as plsc

G = 256

BN = 2000
BE = 8000
BA = 6400


_PREC = lax.Precision.HIGHEST


def _dg(a, b, ca, cb):
    return lax.dot_general(a, b, (((ca,), (cb,)), ((), ())),
                           preferred_element_type=jnp.float32,
                           precision=_PREC)


def _relu(v):
    return jnp.maximum(v, 0.0)


def _enc_node_body(x_ref, w1, b1, w2, b2, o_ref):
    h = _relu(jnp.dot(x_ref[...], w1[...], precision=_PREC,
                      preferred_element_type=jnp.float32) + b1[...])
    o_ref[...] = _relu(jnp.dot(h, w2[...], precision=_PREC,
                               preferred_element_type=jnp.float32) + b2[...])


def _enc_node(x, w1, b1r, w2, b2r):
    n = x.shape[0]
    grid = n // BN
    return pl.pallas_call(
        _enc_node_body,
        grid=(grid,),
        in_specs=[
            pl.BlockSpec((BN, x.shape[1]), lambda i: (i, 0)),
            pl.BlockSpec(w1.shape, lambda i: (0, 0)),
            pl.BlockSpec(b1r.shape, lambda i: (0, 0)),
            pl.BlockSpec(w2.shape, lambda i: (0, 0)),
            pl.BlockSpec(b2r.shape, lambda i: (0, 0)),
        ],
        out_specs=pl.BlockSpec((BN, w2.shape[1]), lambda i: (i, 0)),
        out_shape=jax.ShapeDtypeStruct((n, w2.shape[1]), jnp.float32),
    )(x, w1, b1r, w2, b2r)


def _enc_edge_body(a_ref, w1c, b1c, w2t, b2c, we, ber, o_ref):
    a = a_ref[...]
    h1 = _relu(w1c[...] * a + b1c[...])
    e0 = _relu(_dg(w2t[...], h1, 1, 0) + b2c[...])
    o_ref[...] = _dg(e0, we[...], 0, 0) + ber[...]


def _enc_edge_q1(a_row, w1c, b1c, w2t, b2c, we, ber):
    e = a_row.shape[1]
    grid = e // BA
    return pl.pallas_call(
        _enc_edge_body,
        grid=(grid,),
        in_specs=[
            pl.BlockSpec((1, BA), lambda i: (0, i)),
            pl.BlockSpec(w1c.shape, lambda i: (0, 0)),
            pl.BlockSpec(b1c.shape, lambda i: (0, 0)),
            pl.BlockSpec(w2t.shape, lambda i: (0, 0)),
            pl.BlockSpec(b2c.shape, lambda i: (0, 0)),
            pl.BlockSpec(we.shape, lambda i: (0, 0)),
            pl.BlockSpec(ber.shape, lambda i: (0, 0)),
        ],
        out_specs=pl.BlockSpec((BA, 32), lambda i: (i, 0)),
        out_shape=jax.ShapeDtypeStruct((e, 32), jnp.float32),
    )(a_row, w1c, b1c, w2t, b2c, we, ber)


def _q_body(e_ref, we, ber, o_ref):
    o_ref[...] = jnp.dot(e_ref[...], we[...], precision=_PREC,
                         preferred_element_type=jnp.float32) + ber[...]


def _q_mat(e, we, ber):
    n = e.shape[0]
    grid = n // BE
    return pl.pallas_call(
        _q_body,
        grid=(grid,),
        in_specs=[
            pl.BlockSpec((BE, 32), lambda i: (i, 0)),
            pl.BlockSpec(we.shape, lambda i: (0, 0)),
            pl.BlockSpec(ber.shape, lambda i: (0, 0)),
        ],
        out_specs=pl.BlockSpec((BE, 32), lambda i: (i, 0)),
        out_shape=jax.ShapeDtypeStruct((n, 32), jnp.float32),
    )(e, we, ber)


def _onehot_t(batch_row):
    segs = lax.broadcasted_iota(jnp.int32, (G, batch_row.shape[1]), 0)
    return jnp.where(segs == batch_row, 1.0, 0.0).astype(jnp.float32)


def _p_body_u(x_ref, wx, b3_ref, uproj, o_ref):
    oh = _onehot_t(b3_ref[0])
    ut = _dg(oh, uproj[...], 0, 0)
    o_ref[...] = jnp.dot(x_ref[...], wx[...], precision=_PREC,
                         preferred_element_type=jnp.float32) + ut


def _p_body(x_ref, wx, o_ref):
    o_ref[...] = jnp.dot(x_ref[...], wx[...], precision=_PREC,
                         preferred_element_type=jnp.float32)


def _p_mat(x, wx, batch3, uproj):
    n = x.shape[0]
    grid = n // BN
    if uproj is None:
        return pl.pallas_call(
            _p_body,
            grid=(grid,),
            in_specs=[
                pl.BlockSpec((BN, 128), lambda i: (i, 0)),
                pl.BlockSpec(wx.shape, lambda i: (0, 0)),
            ],
            out_specs=pl.BlockSpec((BN, 32), lambda i: (i, 0)),
            out_shape=jax.ShapeDtypeStruct((n, 32), jnp.float32),
        )(x, wx)
    return pl.pallas_call(
        _p_body_u,
        grid=(grid,),
        in_specs=[
            pl.BlockSpec((BN, 128), lambda i: (i, 0)),
            pl.BlockSpec(wx.shape, lambda i: (0, 0)),
            pl.BlockSpec((1, 1, BN), lambda i: (i, 0, 0)),
            pl.BlockSpec(uproj.shape, lambda i: (0, 0)),
        ],
        out_specs=pl.BlockSpec((BN, 32), lambda i: (i, 0)),
        out_shape=jax.ShapeDtypeStruct((n, 32), jnp.float32),
    )(x, wx, batch3, uproj)


def _node_body(x_ref, m_ref, b3_ref, wx, wm, nbr, x_out, xg_out, *, has_u,
               uproj_ref=None):
    m = m_ref[0] + m_ref[1]
    z = (jnp.dot(x_ref[...], wx[...], precision=_PREC, preferred_element_type=jnp.float32)
         + _dg(m, wm[...], 1, 0) + nbr[...])
    oh = _onehot_t(b3_ref[0])
    if has_u:
        z = z + _dg(oh, uproj_ref[...], 0, 0)
    xn = _relu(z)
    x_out[...] = xn

    @pl.when(pl.program_id(0) == 0)
    def _():
        xg_out[...] = jnp.zeros_like(xg_out)

    xg_out[...] += _dg(oh, xn, 1, 0)


def _node_mat(x, msg_p, batch3, wx, wm, nbr, uprojn):
    n = x.shape[0]
    grid = n // BN
    has_u = uprojn is not None

    if has_u:
        def body(x_ref, m_ref, b3_ref, wx_r, wm_r, nb_r, up_r, x_out, xg_out):
            _node_body(x_ref, m_ref, b3_ref, wx_r, wm_r, nb_r, x_out, xg_out,
                       has_u=True, uproj_ref=up_r)
        extra_specs = [pl.BlockSpec(uprojn.shape, lambda i: (0, 0))]
        args = (x, msg_p, batch3, wx, wm, nbr, uprojn)
    else:
        def body(x_ref, m_ref, b3_ref, wx_r, wm_r, nb_r, x_out, xg_out):
            _node_body(x_ref, m_ref, b3_ref, wx_r, wm_r, nb_r, x_out, xg_out,
                       has_u=False)
        extra_specs = []
        args = (x, msg_p, batch3, wx, wm, nbr)

    return pl.pallas_call(
        body,
        grid=(grid,),
        in_specs=[
            pl.BlockSpec((BN, 128), lambda i: (i, 0)),
            pl.BlockSpec((2, BN, 32), lambda i: (0, i, 0)),
            pl.BlockSpec((1, 1, BN), lambda i: (i, 0, 0)),
            pl.BlockSpec(wx.shape, lambda i: (0, 0)),
            pl.BlockSpec(wm.shape, lambda i: (0, 0)),
            pl.BlockSpec(nbr.shape, lambda i: (0, 0)),
        ] + extra_specs,
        out_specs=[
            pl.BlockSpec((BN, 128), lambda i: (i, 0)),
            pl.BlockSpec((G, 128), lambda i: (0, 0)),
        ],
        out_shape=[
            jax.ShapeDtypeStruct((n, 128), jnp.float32),
            jax.ShapeDtypeStruct((G, 128), jnp.float32),
        ],
    )(*args)


def _global_mat(eg_p, xg, u_prev, we, wxg, wu, gbr,
                eproj_w=None, nproj_w=None, ro_w=None, ro_br=None):
    has_u = u_prev is not None
    has_proj = eproj_w is not None

    in_arrays = [eg_p, xg, we, wxg, gbr]
    if has_u:
        in_arrays += [u_prev, wu]
    out_shape = [jax.ShapeDtypeStruct((G, 32), jnp.float32)]
    if has_proj:
        in_arrays += [eproj_w, nproj_w]
        out_shape += [jax.ShapeDtypeStruct((G, 32), jnp.float32),
                      jax.ShapeDtypeStruct((G, 128), jnp.float32)]
    else:
        in_arrays += [ro_w, ro_br]
        out_shape += [jax.ShapeDtypeStruct((1, G), jnp.float32)]

    def body(*refs):
        k = 5
        egp_r, xg_r, we_r, wxg_r, gb_r = refs[:k]
        if has_u:
            u_r, wu_r = refs[k:k + 2]
            k += 2
        a_r, b_r = refs[k:k + 2]
        k += 2
        outs = refs[k:]
        eg = egp_r[0] + egp_r[1]
        z = (_dg(eg, we_r[...], 1, 0) + _dg(xg_r[...], wxg_r[...], 1, 0)
             + gb_r[...])
        if has_u:
            z = z + _dg(u_r[...], wu_r[...], 1, 0)
        uu = _relu(z)
        outs[0][...] = uu
        if has_proj:
            outs[1][...] = _dg(uu, a_r[...], 1, 0)
            outs[2][...] = _dg(uu, b_r[...], 1, 0)
        else:
            outs[1][...] = _dg(a_r[...], uu, 0, 1) + b_r[...]

    full = lambda a: pl.BlockSpec(a.shape, None)
    return pl.pallas_call(
        body,
        in_specs=[full(a) for a in in_arrays],
        out_specs=[pl.BlockSpec(s.shape, None) for s in out_shape],
        out_shape=out_shape,
    )(*in_arrays)


def _compute_eb(src, batch):
    return batch[src]


def _edge_pass(p, q, src, dst, eb, n):
    e_new = _relu(p[src] + q)
    msg = jax.ops.segment_sum(e_new, dst, num_segments=n)
    eg = jax.ops.segment_sum(e_new, eb, num_segments=G)
    z32 = jnp.zeros_like(msg)
    zg = jnp.zeros_like(eg)
    return e_new, jnp.stack([msg, z32]), jnp.stack([eg, zg])


def kernel(x, edge_index, edge_attr, batch,
           en_W1, en_b1, en_W2, en_b2,
           ee_W1, ee_b1, ee_W2, ee_b2,
           l1_eW, l1_eb, l1_nW, l1_nb, l1_gW, l1_gb,
           l2_eW, l2_eb, l2_nW, l2_nb, l2_gW, l2_gb,
           l3_eW, l3_eb, l3_nW, l3_nb, l3_gW, l3_gb,
           ro_W, ro_b):
    n = x.shape[0]
    e_cnt = edge_index.shape[1]
    src = edge_index[0]
    dst = edge_index[1]
    batch3 = batch.reshape(n // BN, 1, BN)

    row = lambda v: v.reshape(1, -1)
    col = lambda v: v.reshape(-1, 1)

    x0 = _enc_node(x, en_W1, row(en_b1), en_W2, row(en_b2))
    q1 = _enc_edge_q1(edge_attr.reshape(1, e_cnt), col(ee_W1[0]),
                      col(ee_b1), ee_W2.T, col(ee_b2),
                      l1_eW[128:160], row(l1_eb))

    eb_arr = _compute_eb(src, batch)

    layers = [
        (l1_eW, l1_eb, l1_nW, l1_nb, l1_gW, l1_gb, False),
        (l2_eW, l2_eb, l2_nW, l2_nb, l2_gW, l2_gb, True),
        (l3_eW, l3_eb, l3_nW, l3_nb, l3_gW, l3_gb, True),
    ]

    xc = x0
    u = None
    uproj_e = None
    uproj_n = None
    q = q1
    e_arr = None
    res = None

    for li, (eW, ebias, nW, nb, gW, gb, has_u) in enumerate(layers):
        if li > 0:
            q = _q_mat(e_arr, eW[128:160], row(ebias))
        p = _p_mat(xc, eW[:128], batch3, uproj_e if has_u else None)

        e_arr, msg_p, eg_p = _edge_pass(p, q, src, dst, eb_arr, n)

        xc, xg = _node_mat(xc, msg_p, batch3, nW[:128], nW[128:160],
                           row(nb), uproj_n if has_u else None)

        is_last = li == 2
        if not is_last:
            nxt_eW = layers[li + 1][0]
            nxt_nW = layers[li + 1][2]
            outs = _global_mat(eg_p, xg, u, gW[:32], gW[32:160],
                               gW[160:192] if has_u else None, row(gb),
                               eproj_w=nxt_eW[160:192], nproj_w=nxt_nW[160:192])
            u, uproj_e, uproj_n = outs
        else:
            outs = _global_mat(eg_p, xg, u, gW[:32], gW[32:160],
                               gW[160:192], row(gb),
                               ro_w=ro_W, ro_br=row(ro_b))
            res = outs[1]

    return res.reshape(-1)


# speedup vs baseline: 4.4785x; 4.4785x over previous
import functools

import jax
import jax.numpy as jnp
from jax import lax
from jax.experimental import pallas as pl
from jax.experimental.pallas import tpu as pltpu
from jax.experimental.pallas import tpu_sc as plsc

G = 256

BN = 2000
BE = 8000
BA = 6400


_PREC = lax.Precision.HIGHEST


def _dg(a, b, ca, cb):
    return lax.dot_general(a, b, (((ca,), (cb,)), ((), ())),
                           preferred_element_type=jnp.float32,
                           precision=_PREC)


def _relu(v):
    return jnp.maximum(v, 0.0)


def _enc_node_body(x_ref, w1, b1, w2, b2, o_ref):
    h = _relu(jnp.dot(x_ref[...], w1[...], precision=_PREC,
                      preferred_element_type=jnp.float32) + b1[...])
    o_ref[...] = _relu(jnp.dot(h, w2[...], precision=_PREC,
                               preferred_element_type=jnp.float32) + b2[...])


def _enc_node(x, w1, b1r, w2, b2r):
    n = x.shape[0]
    grid = n // BN
    return pl.pallas_call(
        _enc_node_body,
        grid=(grid,),
        in_specs=[
            pl.BlockSpec((BN, x.shape[1]), lambda i: (i, 0)),
            pl.BlockSpec(w1.shape, lambda i: (0, 0)),
            pl.BlockSpec(b1r.shape, lambda i: (0, 0)),
            pl.BlockSpec(w2.shape, lambda i: (0, 0)),
            pl.BlockSpec(b2r.shape, lambda i: (0, 0)),
        ],
        out_specs=pl.BlockSpec((BN, w2.shape[1]), lambda i: (i, 0)),
        out_shape=jax.ShapeDtypeStruct((n, w2.shape[1]), jnp.float32),
    )(x, w1, b1r, w2, b2r)


def _enc_edge_body(a_ref, w1c, b1c, w2t, b2c, we, ber, o_ref):
    a = a_ref[...]
    h1 = _relu(w1c[...] * a + b1c[...])
    e0 = _relu(_dg(w2t[...], h1, 1, 0) + b2c[...])
    o_ref[...] = _dg(e0, we[...], 0, 0) + ber[...]


def _enc_edge_q1(a_row, w1c, b1c, w2t, b2c, we, ber):
    e = a_row.shape[1]
    grid = e // BA
    return pl.pallas_call(
        _enc_edge_body,
        grid=(grid,),
        in_specs=[
            pl.BlockSpec((1, BA), lambda i: (0, i)),
            pl.BlockSpec(w1c.shape, lambda i: (0, 0)),
            pl.BlockSpec(b1c.shape, lambda i: (0, 0)),
            pl.BlockSpec(w2t.shape, lambda i: (0, 0)),
            pl.BlockSpec(b2c.shape, lambda i: (0, 0)),
            pl.BlockSpec(we.shape, lambda i: (0, 0)),
            pl.BlockSpec(ber.shape, lambda i: (0, 0)),
        ],
        out_specs=pl.BlockSpec((BA, 32), lambda i: (i, 0)),
        out_shape=jax.ShapeDtypeStruct((e, 32), jnp.float32),
    )(a_row, w1c, b1c, w2t, b2c, we, ber)


def _q_body(e_ref, we, ber, o_ref):
    o_ref[...] = jnp.dot(e_ref[...], we[...], precision=_PREC,
                         preferred_element_type=jnp.float32) + ber[...]


def _q_mat(e, we, ber):
    n = e.shape[0]
    grid = n // BE
    return pl.pallas_call(
        _q_body,
        grid=(grid,),
        in_specs=[
            pl.BlockSpec((BE, 32), lambda i: (i, 0)),
            pl.BlockSpec(we.shape, lambda i: (0, 0)),
            pl.BlockSpec(ber.shape, lambda i: (0, 0)),
        ],
        out_specs=pl.BlockSpec((BE, 32), lambda i: (i, 0)),
        out_shape=jax.ShapeDtypeStruct((n, 32), jnp.float32),
    )(e, we, ber)


def _onehot_t(batch_row):
    segs = lax.broadcasted_iota(jnp.int32, (G, batch_row.shape[1]), 0)
    return jnp.where(segs == batch_row, 1.0, 0.0).astype(jnp.float32)


def _p_body_u(x_ref, wx, b3_ref, uproj, o_ref):
    oh = _onehot_t(b3_ref[0])
    ut = _dg(oh, uproj[...], 0, 0)
    o_ref[...] = jnp.dot(x_ref[...], wx[...], precision=_PREC,
                         preferred_element_type=jnp.float32) + ut


def _p_body(x_ref, wx, o_ref):
    o_ref[...] = jnp.dot(x_ref[...], wx[...], precision=_PREC,
                         preferred_element_type=jnp.float32)


def _p_mat(x, wx, batch3, uproj):
    n = x.shape[0]
    grid = n // BN
    if uproj is None:
        return pl.pallas_call(
            _p_body,
            grid=(grid,),
            in_specs=[
                pl.BlockSpec((BN, 128), lambda i: (i, 0)),
                pl.BlockSpec(wx.shape, lambda i: (0, 0)),
            ],
            out_specs=pl.BlockSpec((BN, 32), lambda i: (i, 0)),
            out_shape=jax.ShapeDtypeStruct((n, 32), jnp.float32),
        )(x, wx)
    return pl.pallas_call(
        _p_body_u,
        grid=(grid,),
        in_specs=[
            pl.BlockSpec((BN, 128), lambda i: (i, 0)),
            pl.BlockSpec(wx.shape, lambda i: (0, 0)),
            pl.BlockSpec((1, 1, BN), lambda i: (i, 0, 0)),
            pl.BlockSpec(uproj.shape, lambda i: (0, 0)),
        ],
        out_specs=pl.BlockSpec((BN, 32), lambda i: (i, 0)),
        out_shape=jax.ShapeDtypeStruct((n, 32), jnp.float32),
    )(x, wx, batch3, uproj)


def _node_body(x_ref, m_ref, b3_ref, wx, wm, nbr, x_out, xg_out, *, has_u,
               uproj_ref=None):
    m = m_ref[...]
    z = (jnp.dot(x_ref[...], wx[...], precision=_PREC, preferred_element_type=jnp.float32)
         + _dg(m, wm[...], 1, 0) + nbr[...])
    oh = _onehot_t(b3_ref[0])
    if has_u:
        z = z + _dg(oh, uproj_ref[...], 0, 0)
    xn = _relu(z)
    x_out[...] = xn

    @pl.when(pl.program_id(0) == 0)
    def _():
        xg_out[...] = jnp.zeros_like(xg_out)

    xg_out[...] += _dg(oh, xn, 1, 0)


def _node_mat(x, msg_p, batch3, wx, wm, nbr, uprojn):
    n = x.shape[0]
    grid = n // BN
    has_u = uprojn is not None

    if has_u:
        def body(x_ref, m_ref, b3_ref, wx_r, wm_r, nb_r, up_r, x_out, xg_out):
            _node_body(x_ref, m_ref, b3_ref, wx_r, wm_r, nb_r, x_out, xg_out,
                       has_u=True, uproj_ref=up_r)
        extra_specs = [pl.BlockSpec(uprojn.shape, lambda i: (0, 0))]
        args = (x, msg_p, batch3, wx, wm, nbr, uprojn)
    else:
        def body(x_ref, m_ref, b3_ref, wx_r, wm_r, nb_r, x_out, xg_out):
            _node_body(x_ref, m_ref, b3_ref, wx_r, wm_r, nb_r, x_out, xg_out,
                       has_u=False)
        extra_specs = []
        args = (x, msg_p, batch3, wx, wm, nbr)

    return pl.pallas_call(
        body,
        grid=(grid,),
        in_specs=[
            pl.BlockSpec((BN, 128), lambda i: (i, 0)),
            pl.BlockSpec((BN, 32), lambda i: (i, 0)),
            pl.BlockSpec((1, 1, BN), lambda i: (i, 0, 0)),
            pl.BlockSpec(wx.shape, lambda i: (0, 0)),
            pl.BlockSpec(wm.shape, lambda i: (0, 0)),
            pl.BlockSpec(nbr.shape, lambda i: (0, 0)),
        ] + extra_specs,
        out_specs=[
            pl.BlockSpec((BN, 128), lambda i: (i, 0)),
            pl.BlockSpec((G, 128), lambda i: (0, 0)),
        ],
        out_shape=[
            jax.ShapeDtypeStruct((n, 128), jnp.float32),
            jax.ShapeDtypeStruct((G, 128), jnp.float32),
        ],
    )(*args)


def _global_mat(eg_p, xg, u_prev, we, wxg, wu, gbr,
                eproj_w=None, nproj_w=None, ro_w=None, ro_br=None):
    has_u = u_prev is not None
    has_proj = eproj_w is not None

    in_arrays = [eg_p, xg, we, wxg, gbr]
    if has_u:
        in_arrays += [u_prev, wu]
    out_shape = [jax.ShapeDtypeStruct((G, 32), jnp.float32)]
    if has_proj:
        in_arrays += [eproj_w, nproj_w]
        out_shape += [jax.ShapeDtypeStruct((G, 32), jnp.float32),
                      jax.ShapeDtypeStruct((G, 128), jnp.float32)]
    else:
        in_arrays += [ro_w, ro_br]
        out_shape += [jax.ShapeDtypeStruct((1, G), jnp.float32)]

    def body(*refs):
        k = 5
        egp_r, xg_r, we_r, wxg_r, gb_r = refs[:k]
        if has_u:
            u_r, wu_r = refs[k:k + 2]
            k += 2
        a_r, b_r = refs[k:k + 2]
        k += 2
        outs = refs[k:]
        eg = egp_r[0] + egp_r[1]
        z = (_dg(eg, we_r[...], 1, 0) + _dg(xg_r[...], wxg_r[...], 1, 0)
             + gb_r[...])
        if has_u:
            z = z + _dg(u_r[...], wu_r[...], 1, 0)
        uu = _relu(z)
        outs[0][...] = uu
        if has_proj:
            outs[1][...] = _dg(uu, a_r[...], 1, 0)
            outs[2][...] = _dg(uu, b_r[...], 1, 0)
        else:
            outs[1][...] = _dg(a_r[...], uu, 0, 1) + b_r[...]

    full = lambda a: pl.BlockSpec(a.shape, None)
    return pl.pallas_call(
        body,
        in_specs=[full(a) for a in in_arrays],
        out_specs=[pl.BlockSpec(s.shape, None) for s in out_shape],
        out_shape=out_shape,
    )(*in_arrays)


_NW = 32
_EPW = 25000
_CH = 1000
_NCH = _EPW // _CH
_NPAD = 50048
_NPT = _NPAD // 16


def _sc_mesh():
    return plsc.VectorSubcoreMesh(core_axis_name="c", subcore_axis_name="s")


def _compute_eb(src, batch):
    n = batch.shape[0]
    e_cnt = src.shape[0]

    @functools.partial(
        pl.kernel,
        mesh=_sc_mesh(),
        compiler_params=pltpu.CompilerParams(needs_layout_passes=False, use_tc_tiling_on_sc=False),
        out_type=jax.ShapeDtypeStruct((e_cnt,), jnp.int32),
        scratch_types=[
            pltpu.VMEM((n,), jnp.int32),
            pltpu.VMEM((_EPW,), jnp.int32),
            pltpu.VMEM((_EPW,), jnp.int32),
        ],
    )
    def k(src_hbm, batch_hbm, eb_out, batch_v, src_v, eb_v):
        wid = lax.axis_index("s") * 2 + lax.axis_index("c")
        base = wid * _EPW
        pltpu.sync_copy(batch_hbm, batch_v)
        pltpu.sync_copy(src_hbm.at[pl.ds(base, _EPW)], src_v)

        def body(j, _):
            off = jnp.minimum(j * 16, _EPW - 16)
            idx = src_v[pl.ds(off, 16)]
            eb_v[pl.ds(off, 16)] = plsc.load_gather(batch_v, [idx])
            return 0

        lax.fori_loop(0, (_EPW + 15) // 16, body, 0)
        pltpu.sync_copy(eb_v, eb_out.at[pl.ds(base, _EPW)])

    return k(src, batch)


def _edge_pass(p, q, src, eb):
    e_cnt = q.shape[0]

    @functools.partial(
        pl.kernel,
        mesh=_sc_mesh(),
        compiler_params=pltpu.CompilerParams(needs_layout_passes=False,
                                             use_tc_tiling_on_sc=False),
        out_type=[
            jax.ShapeDtypeStruct((e_cnt, 32), jnp.float32),
            jax.ShapeDtypeStruct((2, G, 32), jnp.float32),
        ],
        scratch_types=[
            pltpu.VMEM((_CH,), jnp.int32),
            pltpu.VMEM((_CH,), jnp.int32),
            pltpu.VMEM((_CH, 32), jnp.float32),
            pltpu.VMEM((_CH, 32), jnp.float32),
            pltpu.VMEM((16, 32), jnp.float32),
            pltpu.VMEM_SHARED((G, 32), jnp.float32),
            pltpu.SemaphoreType.DMA,
        ],
    )
    def ka(p_hbm, q_hbm, src_hbm, eb_hbm, e_out, eg_out,
           src_v, eb_v, q_v, pg_v, zb_v, eg_sh, sem):
        c = lax.axis_index("c")
        s = lax.axis_index("s")
        wid = s * 2 + c
        zero16 = jnp.zeros((16,), jnp.float32)

        def zfill(i, _):
            zb_v[i, pl.ds(0, 16)] = zero16
            zb_v[i, pl.ds(16, 16)] = zero16
            return 0

        lax.fori_loop(0, 16, zfill, 0)
        pltpu.sync_copy(zb_v, eg_sh.at[pl.ds(s * 16, 16)])
        plsc.subcore_barrier()

        base = wid * _EPW

        def chunk(kk, _):
            off = base + kk * _CH
            pltpu.sync_copy(src_hbm.at[pl.ds(off, _CH)], src_v)
            pltpu.sync_copy(q_hbm.at[pl.ds(off, _CH)], q_v)
            pltpu.async_copy(p_hbm.at[src_v], pg_v, sem).wait()

            def rows(r, _):
                a0 = pg_v[r, pl.ds(0, 16)] + q_v[r, pl.ds(0, 16)]
                a1 = pg_v[r, pl.ds(16, 16)] + q_v[r, pl.ds(16, 16)]
                q_v[r, pl.ds(0, 16)] = jnp.maximum(a0, 0.0)
                q_v[r, pl.ds(16, 16)] = jnp.maximum(a1, 0.0)
                return 0

            lax.fori_loop(0, _CH, rows, 0)
            pltpu.sync_copy(q_v, e_out.at[pl.ds(off, _CH)])
            pltpu.sync_copy(eb_hbm.at[pl.ds(off, _CH)], eb_v)
            pltpu.sync_copy(q_v, eg_sh.at[eb_v], add=True)
            return 0

        lax.fori_loop(0, _NCH, chunk, 0)
        plsc.subcore_barrier()
        pltpu.sync_copy(eg_sh.at[pl.ds(s * 16, 16)],
                        eg_out.at[c, pl.ds(s * 16, 16)])

    return ka(p, q, src, eb)


_QR = _NPAD // 4
_QPT = _QR // 16
_DUMP = 2048
_CHB = 2000
_EPT_B = 800000 // 16


def _msg_pass(dst, e_new):
    @functools.partial(
        pl.kernel,
        mesh=_sc_mesh(),
        compiler_params=pltpu.CompilerParams(needs_layout_passes=False,
                                             use_tc_tiling_on_sc=False),
        out_type=jax.ShapeDtypeStruct((_NPAD, 32), jnp.float32),
        scratch_types=[
            pltpu.VMEM((_CHB,), jnp.int32),
            pltpu.VMEM((_CHB,), jnp.int32),
            pltpu.VMEM((_CHB, 32), jnp.float32),
            pltpu.VMEM((_QPT, 32), jnp.float32),
            pltpu.VMEM_SHARED((_QR + _DUMP, 32), jnp.float32),
            pltpu.SemaphoreType.DMA,
        ],
    )
    def kb(dst_hbm, e_hbm, msg_out, d_v, iq_v, rows_v, zb_v, acc, sem):
        c = lax.axis_index("c")
        s = lax.axis_index("s")
        zero16 = jnp.zeros((16,), jnp.float32)

        def zfill(i, _):
            zb_v[i, pl.ds(0, 16)] = zero16
            zb_v[i, pl.ds(16, 16)] = zero16
            return 0

        lax.fori_loop(0, _QPT, zfill, 0)

        for qq in range(2):
            qlo = (2 * c + qq) * _QR
            pltpu.sync_copy(zb_v, acc.at[pl.ds(s * _QPT, _QPT)])
            for t in range(_DUMP // _QPT + 1):
                lo = jnp.minimum(_QR + t * _QPT, _QR + _DUMP - _QPT)
                pltpu.sync_copy(zb_v, acc.at[pl.ds(lo, _QPT)])
            plsc.subcore_barrier()

            def chunk(kk, _):
                off = s * _EPT_B + kk * _CHB
                pltpu.sync_copy(dst_hbm.at[pl.ds(off, _CHB)], d_v)
                pltpu.sync_copy(e_hbm.at[pl.ds(off, _CHB)], rows_v)

                def vec(i, _):
                    d = d_v[pl.ds(i * 16, 16)]
                    dq = d - qlo
                    m = (dq >= 0) & (dq < _QR)
                    sp = _QR + ((kk * _CHB + i * 16
                                 + lax.broadcasted_iota(jnp.int32, (16,), 0))
                                & (_DUMP - 1))
                    iq_v[pl.ds(i * 16, 16)] = jnp.where(m, dq, sp)
                    return 0

                lax.fori_loop(0, _CHB // 16, vec, 0)
                pltpu.sync_copy(rows_v, acc.at[iq_v], add=True)
                return 0

            lax.fori_loop(0, _EPT_B // _CHB, chunk, 0)
            plsc.subcore_barrier()
            pltpu.sync_copy(acc.at[pl.ds(s * _QPT, _QPT)],
                            msg_out.at[pl.ds(qlo + s * _QPT, _QPT)])
            plsc.subcore_barrier()

    return kb(dst, e_new)


def kernel(x, edge_index, edge_attr, batch,
           en_W1, en_b1, en_W2, en_b2,
           ee_W1, ee_b1, ee_W2, ee_b2,
           l1_eW, l1_eb, l1_nW, l1_nb, l1_gW, l1_gb,
           l2_eW, l2_eb, l2_nW, l2_nb, l2_gW, l2_gb,
           l3_eW, l3_eb, l3_nW, l3_nb, l3_gW, l3_gb,
           ro_W, ro_b):
    n = x.shape[0]
    e_cnt = edge_index.shape[1]
    src = edge_index[0]
    dst = edge_index[1]
    batch3 = batch.reshape(n // BN, 1, BN)

    row = lambda v: v.reshape(1, -1)
    col = lambda v: v.reshape(-1, 1)

    x0 = _enc_node(x, en_W1, row(en_b1), en_W2, row(en_b2))
    q1 = _enc_edge_q1(edge_attr.reshape(1, e_cnt), col(ee_W1[0]),
                      col(ee_b1), ee_W2.T, col(ee_b2),
                      l1_eW[128:160], row(l1_eb))

    eb_arr = _compute_eb(src, batch)

    layers = [
        (l1_eW, l1_eb, l1_nW, l1_nb, l1_gW, l1_gb, False),
        (l2_eW, l2_eb, l2_nW, l2_nb, l2_gW, l2_gb, True),
        (l3_eW, l3_eb, l3_nW, l3_nb, l3_gW, l3_gb, True),
    ]

    xc = x0
    u = None
    uproj_e = None
    uproj_n = None
    q = q1
    e_arr = None
    res = None

    for li, (eW, ebias, nW, nb, gW, gb, has_u) in enumerate(layers):
        if li > 0:
            q = _q_mat(e_arr, eW[128:160], row(ebias))
        p = _p_mat(xc, eW[:128], batch3, uproj_e if has_u else None)

        e_arr, eg_p = _edge_pass(p, q, src, eb_arr)
        msg = _msg_pass(dst, e_arr)

        xc, xg = _node_mat(xc, msg, batch3, nW[:128], nW[128:160],
                           row(nb), uproj_n if has_u else None)

        is_last = li == 2
        if not is_last:
            nxt_eW = layers[li + 1][0]
            nxt_nW = layers[li + 1][2]
            outs = _global_mat(eg_p, xg, u, gW[:32], gW[32:160],
                               gW[160:192] if has_u else None, row(gb),
                               eproj_w=nxt_eW[160:192], nproj_w=nxt_nW[160:192])
            u, uproj_e, uproj_n = outs
        else:
            outs = _global_mat(eg_p, xg, u, gW[:32], gW[32:160],
                               gW[160:192], row(gb),
                               ro_w=ro_W, ro_br=row(ro_b))
            res = outs[1]

    return res.reshape(-1)


# speedup vs baseline: 4.6721x; 1.0432x over previous
import functools

import jax
import jax.numpy as jnp
from jax import lax
from jax.experimental import pallas as pl
from jax.experimental.pallas import tpu as pltpu
from jax.experimental.pallas import tpu_sc as plsc

G = 256

BN = 2000
BE = 8000
BA = 6400


_PREC = lax.Precision.HIGHEST


def _dg(a, b, ca, cb):
    return lax.dot_general(a, b, (((ca,), (cb,)), ((), ())),
                           preferred_element_type=jnp.float32,
                           precision=_PREC)


def _relu(v):
    return jnp.maximum(v, 0.0)


def _enc_node_body(x_ref, w1, b1, w2, b2, o_ref):
    h = _relu(jnp.dot(x_ref[...], w1[...], precision=_PREC,
                      preferred_element_type=jnp.float32) + b1[...])
    o_ref[...] = _relu(jnp.dot(h, w2[...], precision=_PREC,
                               preferred_element_type=jnp.float32) + b2[...])


def _enc_node(x, w1, b1r, w2, b2r):
    n = x.shape[0]
    grid = n // BN
    return pl.pallas_call(
        _enc_node_body,
        grid=(grid,),
        in_specs=[
            pl.BlockSpec((BN, x.shape[1]), lambda i: (i, 0)),
            pl.BlockSpec(w1.shape, lambda i: (0, 0)),
            pl.BlockSpec(b1r.shape, lambda i: (0, 0)),
            pl.BlockSpec(w2.shape, lambda i: (0, 0)),
            pl.BlockSpec(b2r.shape, lambda i: (0, 0)),
        ],
        out_specs=pl.BlockSpec((BN, w2.shape[1]), lambda i: (i, 0)),
        out_shape=jax.ShapeDtypeStruct((n, w2.shape[1]), jnp.float32),
    )(x, w1, b1r, w2, b2r)


def _enc_edge_body(a_ref, w1c, b1c, w2t, b2c, we, ber, o_ref):
    a = a_ref[...]
    h1 = _relu(w1c[...] * a + b1c[...])
    e0 = _relu(_dg(w2t[...], h1, 1, 0) + b2c[...])
    o_ref[...] = _dg(e0, we[...], 0, 0) + ber[...]


def _enc_edge_q1(a_row, w1c, b1c, w2t, b2c, we, ber):
    e = a_row.shape[1]
    grid = e // BA
    return pl.pallas_call(
        _enc_edge_body,
        grid=(grid,),
        in_specs=[
            pl.BlockSpec((1, BA), lambda i: (0, i)),
            pl.BlockSpec(w1c.shape, lambda i: (0, 0)),
            pl.BlockSpec(b1c.shape, lambda i: (0, 0)),
            pl.BlockSpec(w2t.shape, lambda i: (0, 0)),
            pl.BlockSpec(b2c.shape, lambda i: (0, 0)),
            pl.BlockSpec(we.shape, lambda i: (0, 0)),
            pl.BlockSpec(ber.shape, lambda i: (0, 0)),
        ],
        out_specs=pl.BlockSpec((BA, 32), lambda i: (i, 0)),
        out_shape=jax.ShapeDtypeStruct((e, 32), jnp.float32),
    )(a_row, w1c, b1c, w2t, b2c, we, ber)


def _q_body(e_ref, we, ber, o_ref):
    o_ref[...] = jnp.dot(e_ref[...], we[...], precision=_PREC,
                         preferred_element_type=jnp.float32) + ber[...]


def _q_mat(e, we, ber):
    n = e.shape[0]
    grid = n // BE
    return pl.pallas_call(
        _q_body,
        grid=(grid,),
        in_specs=[
            pl.BlockSpec((BE, 32), lambda i: (i, 0)),
            pl.BlockSpec(we.shape, lambda i: (0, 0)),
            pl.BlockSpec(ber.shape, lambda i: (0, 0)),
        ],
        out_specs=pl.BlockSpec((BE, 32), lambda i: (i, 0)),
        out_shape=jax.ShapeDtypeStruct((n, 32), jnp.float32),
    )(e, we, ber)


def _onehot_t(batch_row):
    segs = lax.broadcasted_iota(jnp.int32, (G, batch_row.shape[1]), 0)
    return jnp.where(segs == batch_row, 1.0, 0.0).astype(jnp.float32)


def _p_body_u(x_ref, wx, b3_ref, uproj, o_ref):
    oh = _onehot_t(b3_ref[0])
    ut = _dg(oh, uproj[...], 0, 0)
    o_ref[...] = jnp.dot(x_ref[...], wx[...], precision=_PREC,
                         preferred_element_type=jnp.float32) + ut


def _p_body(x_ref, wx, o_ref):
    o_ref[...] = jnp.dot(x_ref[...], wx[...], precision=_PREC,
                         preferred_element_type=jnp.float32)


def _p_mat(x, wx, batch3, uproj):
    n = x.shape[0]
    grid = n // BN
    if uproj is None:
        return pl.pallas_call(
            _p_body,
            grid=(grid,),
            in_specs=[
                pl.BlockSpec((BN, 128), lambda i: (i, 0)),
                pl.BlockSpec(wx.shape, lambda i: (0, 0)),
            ],
            out_specs=pl.BlockSpec((BN, 32), lambda i: (i, 0)),
            out_shape=jax.ShapeDtypeStruct((n, 32), jnp.float32),
        )(x, wx)
    return pl.pallas_call(
        _p_body_u,
        grid=(grid,),
        in_specs=[
            pl.BlockSpec((BN, 128), lambda i: (i, 0)),
            pl.BlockSpec(wx.shape, lambda i: (0, 0)),
            pl.BlockSpec((1, 1, BN), lambda i: (i, 0, 0)),
            pl.BlockSpec(uproj.shape, lambda i: (0, 0)),
        ],
        out_specs=pl.BlockSpec((BN, 32), lambda i: (i, 0)),
        out_shape=jax.ShapeDtypeStruct((n, 32), jnp.float32),
    )(x, wx, batch3, uproj)


def _node_body(x_ref, m_ref, b3_ref, wx, wm, nbr, x_out, xg_out, *, has_u,
               uproj_ref=None):
    m = m_ref[...]
    z = (jnp.dot(x_ref[...], wx[...], precision=_PREC, preferred_element_type=jnp.float32)
         + _dg(m, wm[...], 1, 0) + nbr[...])
    oh = _onehot_t(b3_ref[0])
    if has_u:
        z = z + _dg(oh, uproj_ref[...], 0, 0)
    xn = _relu(z)
    x_out[...] = xn

    @pl.when(pl.program_id(0) == 0)
    def _():
        xg_out[...] = jnp.zeros_like(xg_out)

    xg_out[...] += _dg(oh, xn, 1, 0)


def _node_mat(x, msg_p, batch3, wx, wm, nbr, uprojn):
    n = x.shape[0]
    grid = n // BN
    has_u = uprojn is not None

    if has_u:
        def body(x_ref, m_ref, b3_ref, wx_r, wm_r, nb_r, up_r, x_out, xg_out):
            _node_body(x_ref, m_ref, b3_ref, wx_r, wm_r, nb_r, x_out, xg_out,
                       has_u=True, uproj_ref=up_r)
        extra_specs = [pl.BlockSpec(uprojn.shape, lambda i: (0, 0))]
        args = (x, msg_p, batch3, wx, wm, nbr, uprojn)
    else:
        def body(x_ref, m_ref, b3_ref, wx_r, wm_r, nb_r, x_out, xg_out):
            _node_body(x_ref, m_ref, b3_ref, wx_r, wm_r, nb_r, x_out, xg_out,
                       has_u=False)
        extra_specs = []
        args = (x, msg_p, batch3, wx, wm, nbr)

    return pl.pallas_call(
        body,
        grid=(grid,),
        in_specs=[
            pl.BlockSpec((BN, 128), lambda i: (i, 0)),
            pl.BlockSpec((BN, 32), lambda i: (i, 0)),
            pl.BlockSpec((1, 1, BN), lambda i: (i, 0, 0)),
            pl.BlockSpec(wx.shape, lambda i: (0, 0)),
            pl.BlockSpec(wm.shape, lambda i: (0, 0)),
            pl.BlockSpec(nbr.shape, lambda i: (0, 0)),
        ] + extra_specs,
        out_specs=[
            pl.BlockSpec((BN, 128), lambda i: (i, 0)),
            pl.BlockSpec((G, 128), lambda i: (0, 0)),
        ],
        out_shape=[
            jax.ShapeDtypeStruct((n, 128), jnp.float32),
            jax.ShapeDtypeStruct((G, 128), jnp.float32),
        ],
    )(*args)


def _global_mat(eg_p, xg, u_prev, we, wxg, wu, gbr,
                eproj_w=None, nproj_w=None, ro_w=None, ro_br=None):
    has_u = u_prev is not None
    has_proj = eproj_w is not None

    in_arrays = [eg_p, xg, we, wxg, gbr]
    if has_u:
        in_arrays += [u_prev, wu]
    out_shape = [jax.ShapeDtypeStruct((G, 32), jnp.float32)]
    if has_proj:
        in_arrays += [eproj_w, nproj_w]
        out_shape += [jax.ShapeDtypeStruct((G, 32), jnp.float32),
                      jax.ShapeDtypeStruct((G, 128), jnp.float32)]
    else:
        in_arrays += [ro_w, ro_br]
        out_shape += [jax.ShapeDtypeStruct((1, G), jnp.float32)]

    def body(*refs):
        k = 5
        egp_r, xg_r, we_r, wxg_r, gb_r = refs[:k]
        if has_u:
            u_r, wu_r = refs[k:k + 2]
            k += 2
        a_r, b_r = refs[k:k + 2]
        k += 2
        outs = refs[k:]
        eg = egp_r[0] + egp_r[1]
        z = (_dg(eg, we_r[...], 1, 0) + _dg(xg_r[...], wxg_r[...], 1, 0)
             + gb_r[...])
        if has_u:
            z = z + _dg(u_r[...], wu_r[...], 1, 0)
        uu = _relu(z)
        outs[0][...] = uu
        if has_proj:
            outs[1][...] = _dg(uu, a_r[...], 1, 0)
            outs[2][...] = _dg(uu, b_r[...], 1, 0)
        else:
            outs[1][...] = _dg(a_r[...], uu, 0, 1) + b_r[...]

    full = lambda a: pl.BlockSpec(a.shape, None)
    return pl.pallas_call(
        body,
        in_specs=[full(a) for a in in_arrays],
        out_specs=[pl.BlockSpec(s.shape, None) for s in out_shape],
        out_shape=out_shape,
    )(*in_arrays)


_NW = 32
_EPW = 25000
_CH = 1000
_NCH = _EPW // _CH
_NPAD = 50048
_NPT = _NPAD // 16


def _sc_mesh():
    return plsc.VectorSubcoreMesh(core_axis_name="c", subcore_axis_name="s")


def _compute_eb(src, batch):
    n = batch.shape[0]
    e_cnt = src.shape[0]

    @functools.partial(
        pl.kernel,
        mesh=_sc_mesh(),
        compiler_params=pltpu.CompilerParams(needs_layout_passes=False, use_tc_tiling_on_sc=False),
        out_type=jax.ShapeDtypeStruct((e_cnt,), jnp.int32),
        scratch_types=[
            pltpu.VMEM((n,), jnp.int32),
            pltpu.VMEM((_EPW,), jnp.int32),
            pltpu.VMEM((_EPW,), jnp.int32),
        ],
    )
    def k(src_hbm, batch_hbm, eb_out, batch_v, src_v, eb_v):
        wid = lax.axis_index("s") * 2 + lax.axis_index("c")
        base = wid * _EPW
        pltpu.sync_copy(batch_hbm, batch_v)
        pltpu.sync_copy(src_hbm.at[pl.ds(base, _EPW)], src_v)

        def body(j, _):
            off = jnp.minimum(j * 16, _EPW - 16)
            idx = src_v[pl.ds(off, 16)]
            eb_v[pl.ds(off, 16)] = plsc.load_gather(batch_v, [idx])
            return 0

        lax.fori_loop(0, (_EPW + 15) // 16, body, 0)
        pltpu.sync_copy(eb_v, eb_out.at[pl.ds(base, _EPW)])

    return k(src, batch)


def _edge_pass(p, q, src, eb):
    e_cnt = q.shape[0]

    @functools.partial(
        pl.kernel,
        mesh=_sc_mesh(),
        compiler_params=pltpu.CompilerParams(needs_layout_passes=False,
                                             use_tc_tiling_on_sc=False),
        out_type=[
            jax.ShapeDtypeStruct((e_cnt, 32), jnp.float32),
            jax.ShapeDtypeStruct((2, G, 32), jnp.float32),
        ],
        scratch_types=[
            pltpu.VMEM((_CH,), jnp.int32),
            pltpu.VMEM((_CH,), jnp.int32),
            pltpu.VMEM((_CH, 32), jnp.float32),
            pltpu.VMEM((_CH, 32), jnp.float32),
            pltpu.VMEM((16, 32), jnp.float32),
            pltpu.VMEM_SHARED((G, 32), jnp.float32),
            pltpu.SemaphoreType.DMA,
        ],
    )
    def ka(p_hbm, q_hbm, src_hbm, eb_hbm, e_out, eg_out,
           src_v, eb_v, q_v, pg_v, zb_v, eg_sh, sem):
        c = lax.axis_index("c")
        s = lax.axis_index("s")
        wid = s * 2 + c
        zero16 = jnp.zeros((16,), jnp.float32)

        def zfill(i, _):
            zb_v[i, pl.ds(0, 16)] = zero16
            zb_v[i, pl.ds(16, 16)] = zero16
            return 0

        lax.fori_loop(0, 16, zfill, 0)
        pltpu.sync_copy(zb_v, eg_sh.at[pl.ds(s * 16, 16)])
        plsc.subcore_barrier()

        base = wid * _EPW

        def chunk(kk, _):
            off = base + kk * _CH
            pltpu.sync_copy(src_hbm.at[pl.ds(off, _CH)], src_v)
            pltpu.sync_copy(q_hbm.at[pl.ds(off, _CH)], q_v)
            pltpu.async_copy(p_hbm.at[src_v], pg_v, sem).wait()

            def rows(rr, _):
                for u in range(4):
                    r = rr * 4 + u
                    a0 = pg_v[r, pl.ds(0, 16)] + q_v[r, pl.ds(0, 16)]
                    a1 = pg_v[r, pl.ds(16, 16)] + q_v[r, pl.ds(16, 16)]
                    q_v[r, pl.ds(0, 16)] = jnp.maximum(a0, 0.0)
                    q_v[r, pl.ds(16, 16)] = jnp.maximum(a1, 0.0)
                return 0

            lax.fori_loop(0, _CH // 4, rows, 0)
            pltpu.sync_copy(q_v, e_out.at[pl.ds(off, _CH)])
            pltpu.sync_copy(eb_hbm.at[pl.ds(off, _CH)], eb_v)
            pltpu.sync_copy(q_v, eg_sh.at[eb_v], add=True)
            return 0

        lax.fori_loop(0, _NCH, chunk, 0)
        plsc.subcore_barrier()
        pltpu.sync_copy(eg_sh.at[pl.ds(s * 16, 16)],
                        eg_out.at[c, pl.ds(s * 16, 16)])

    return ka(p, q, src, eb)


_QR = _NPAD // 4
_QPT = _QR // 16
_DUMP = 2048
_CHB = 2000
_EPT_B = 800000 // 16


def _msg_pass(dst, e_new):
    @functools.partial(
        pl.kernel,
        mesh=_sc_mesh(),
        compiler_params=pltpu.CompilerParams(needs_layout_passes=False,
                                             use_tc_tiling_on_sc=False),
        out_type=jax.ShapeDtypeStruct((_NPAD, 32), jnp.float32),
        scratch_types=[
            pltpu.VMEM((_CHB,), jnp.int32),
            pltpu.VMEM((_CHB,), jnp.int32),
            pltpu.VMEM((_CHB, 32), jnp.float32),
            pltpu.VMEM((_QPT, 32), jnp.float32),
            pltpu.VMEM_SHARED((_QR + _DUMP, 32), jnp.float32),
            pltpu.SemaphoreType.DMA,
        ],
    )
    def kb(dst_hbm, e_hbm, msg_out, d_v, iq_v, rows_v, zb_v, acc, sem):
        c = lax.axis_index("c")
        s = lax.axis_index("s")
        zero16 = jnp.zeros((16,), jnp.float32)

        def zfill(i, _):
            zb_v[i, pl.ds(0, 16)] = zero16
            zb_v[i, pl.ds(16, 16)] = zero16
            return 0

        lax.fori_loop(0, _QPT, zfill, 0)

        for qq in range(2):
            qlo = (2 * c + qq) * _QR
            pltpu.sync_copy(zb_v, acc.at[pl.ds(s * _QPT, _QPT)])
            for t in range(_DUMP // _QPT + 1):
                lo = jnp.minimum(_QR + t * _QPT, _QR + _DUMP - _QPT)
                pltpu.sync_copy(zb_v, acc.at[pl.ds(lo, _QPT)])
            plsc.subcore_barrier()

            def chunk(kk, _):
                off = s * _EPT_B + kk * _CHB
                pltpu.sync_copy(dst_hbm.at[pl.ds(off, _CHB)], d_v)
                pltpu.sync_copy(e_hbm.at[pl.ds(off, _CHB)], rows_v)

                def vec(ii, _):
                    for u in range(5):
                        i = ii * 5 + u
                        d = d_v[pl.ds(i * 16, 16)]
                        dq = d - qlo
                        m = (dq >= 0) & (dq < _QR)
                        sp = _QR + ((kk * _CHB + i * 16
                                     + lax.broadcasted_iota(jnp.int32, (16,), 0))
                                    & (_DUMP - 1))
                        iq_v[pl.ds(i * 16, 16)] = jnp.where(m, dq, sp)
                    return 0

                lax.fori_loop(0, _CHB // 80, vec, 0)
                pltpu.sync_copy(rows_v, acc.at[iq_v], add=True)
                return 0

            lax.fori_loop(0, _EPT_B // _CHB, chunk, 0)
            plsc.subcore_barrier()
            pltpu.sync_copy(acc.at[pl.ds(s * _QPT, _QPT)],
                            msg_out.at[pl.ds(qlo + s * _QPT, _QPT)])
            plsc.subcore_barrier()

    return kb(dst, e_new)


def kernel(x, edge_index, edge_attr, batch,
           en_W1, en_b1, en_W2, en_b2,
           ee_W1, ee_b1, ee_W2, ee_b2,
           l1_eW, l1_eb, l1_nW, l1_nb, l1_gW, l1_gb,
           l2_eW, l2_eb, l2_nW, l2_nb, l2_gW, l2_gb,
           l3_eW, l3_eb, l3_nW, l3_nb, l3_gW, l3_gb,
           ro_W, ro_b):
    n = x.shape[0]
    e_cnt = edge_index.shape[1]
    src = edge_index[0]
    dst = edge_index[1]
    batch3 = batch.reshape(n // BN, 1, BN)

    row = lambda v: v.reshape(1, -1)
    col = lambda v: v.reshape(-1, 1)

    x0 = _enc_node(x, en_W1, row(en_b1), en_W2, row(en_b2))
    q1 = _enc_edge_q1(edge_attr.reshape(1, e_cnt), col(ee_W1[0]),
                      col(ee_b1), ee_W2.T, col(ee_b2),
                      l1_eW[128:160], row(l1_eb))

    eb_arr = _compute_eb(src, batch)

    layers = [
        (l1_eW, l1_eb, l1_nW, l1_nb, l1_gW, l1_gb, False),
        (l2_eW, l2_eb, l2_nW, l2_nb, l2_gW, l2_gb, True),
        (l3_eW, l3_eb, l3_nW, l3_nb, l3_gW, l3_gb, True),
    ]

    xc = x0
    u = None
    uproj_e = None
    uproj_n = None
    q = q1
    e_arr = None
    res = None

    for li, (eW, ebias, nW, nb, gW, gb, has_u) in enumerate(layers):
        if li > 0:
            q = _q_mat(e_arr, eW[128:160], row(ebias))
        p = _p_mat(xc, eW[:128], batch3, uproj_e if has_u else None)

        e_arr, eg_p = _edge_pass(p, q, src, eb_arr)
        msg = _msg_pass(dst, e_arr)

        xc, xg = _node_mat(xc, msg, batch3, nW[:128], nW[128:160],
                           row(nb), uproj_n if has_u else None)

        is_last = li == 2
        if not is_last:
            nxt_eW = layers[li + 1][0]
            nxt_nW = layers[li + 1][2]
            outs = _global_mat(eg_p, xg, u, gW[:32], gW[32:160],
                               gW[160:192] if has_u else None, row(gb),
                               eproj_w=nxt_eW[160:192], nproj_w=nxt_nW[160:192])
            u, uproj_e, uproj_n = outs
        else:
            outs = _global_mat(eg_p, xg, u, gW[:32], gW[32:160],
                               gW[160:192], row(gb),
                               ro_w=ro_W, ro_br=row(ro_b))
            res = outs[1]

    return res.reshape(-1)


# speedup vs baseline: 4.8422x; 1.0364x over previous
import functools

import jax
import jax.numpy as jnp
from jax import lax
from jax.experimental import pallas as pl
from jax.experimental.pallas import tpu as pltpu
from jax.experimental.pallas import tpu_sc as plsc

G = 256

BN = 2000
BE = 8000
BA = 6400


_PREC = lax.Precision.HIGHEST


def _dg(a, b, ca, cb):
    return lax.dot_general(a, b, (((ca,), (cb,)), ((), ())),
                           preferred_element_type=jnp.float32,
                           precision=_PREC)


def _relu(v):
    return jnp.maximum(v, 0.0)


def _enc_node_body(x_ref, w1, b1, w2, b2, o_ref):
    h = _relu(jnp.dot(x_ref[...], w1[...], precision=_PREC,
                      preferred_element_type=jnp.float32) + b1[...])
    o_ref[...] = _relu(jnp.dot(h, w2[...], precision=_PREC,
                               preferred_element_type=jnp.float32) + b2[...])


def _enc_node(x, w1, b1r, w2, b2r):
    n = x.shape[0]
    grid = n // BN
    return pl.pallas_call(
        _enc_node_body,
        grid=(grid,),
        in_specs=[
            pl.BlockSpec((BN, x.shape[1]), lambda i: (i, 0)),
            pl.BlockSpec(w1.shape, lambda i: (0, 0)),
            pl.BlockSpec(b1r.shape, lambda i: (0, 0)),
            pl.BlockSpec(w2.shape, lambda i: (0, 0)),
            pl.BlockSpec(b2r.shape, lambda i: (0, 0)),
        ],
        out_specs=pl.BlockSpec((BN, w2.shape[1]), lambda i: (i, 0)),
        out_shape=jax.ShapeDtypeStruct((n, w2.shape[1]), jnp.float32),
    )(x, w1, b1r, w2, b2r)


def _enc_edge_body(a_ref, w1c, b1c, w2t, b2c, we, ber, o_ref):
    a = a_ref[...]
    h1 = _relu(w1c[...] * a + b1c[...])
    e0 = _relu(_dg(w2t[...], h1, 1, 0) + b2c[...])
    o_ref[...] = _dg(e0, we[...], 0, 0) + ber[...]


def _enc_edge_q1(a_row, w1c, b1c, w2t, b2c, we, ber):
    e = a_row.shape[1]
    grid = e // BA
    return pl.pallas_call(
        _enc_edge_body,
        grid=(grid,),
        in_specs=[
            pl.BlockSpec((1, BA), lambda i: (0, i)),
            pl.BlockSpec(w1c.shape, lambda i: (0, 0)),
            pl.BlockSpec(b1c.shape, lambda i: (0, 0)),
            pl.BlockSpec(w2t.shape, lambda i: (0, 0)),
            pl.BlockSpec(b2c.shape, lambda i: (0, 0)),
            pl.BlockSpec(we.shape, lambda i: (0, 0)),
            pl.BlockSpec(ber.shape, lambda i: (0, 0)),
        ],
        out_specs=pl.BlockSpec((BA, 32), lambda i: (i, 0)),
        out_shape=jax.ShapeDtypeStruct((e, 32), jnp.float32),
    )(a_row, w1c, b1c, w2t, b2c, we, ber)


def _q_body(e_ref, we, ber, o_ref):
    o_ref[...] = jnp.dot(e_ref[...], we[...], precision=_PREC,
                         preferred_element_type=jnp.float32) + ber[...]


def _q_mat(e, we, ber):
    n = e.shape[0]
    grid = n // BE
    return pl.pallas_call(
        _q_body,
        grid=(grid,),
        in_specs=[
            pl.BlockSpec((BE, 32), lambda i: (i, 0)),
            pl.BlockSpec(we.shape, lambda i: (0, 0)),
            pl.BlockSpec(ber.shape, lambda i: (0, 0)),
        ],
        out_specs=pl.BlockSpec((BE, 32), lambda i: (i, 0)),
        out_shape=jax.ShapeDtypeStruct((n, 32), jnp.float32),
    )(e, we, ber)


def _onehot_t(batch_row):
    segs = lax.broadcasted_iota(jnp.int32, (G, batch_row.shape[1]), 0)
    return jnp.where(segs == batch_row, 1.0, 0.0).astype(jnp.float32)


def _p_body_u(x_ref, wx, b3_ref, uproj, o_ref):
    oh = _onehot_t(b3_ref[0])
    ut = _dg(oh, uproj[...], 0, 0)
    o_ref[...] = jnp.dot(x_ref[...], wx[...], precision=_PREC,
                         preferred_element_type=jnp.float32) + ut


def _p_body(x_ref, wx, o_ref):
    o_ref[...] = jnp.dot(x_ref[...], wx[...], precision=_PREC,
                         preferred_element_type=jnp.float32)


def _p_mat(x, wx, batch3, uproj):
    n = x.shape[0]
    grid = n // BN
    if uproj is None:
        return pl.pallas_call(
            _p_body,
            grid=(grid,),
            in_specs=[
                pl.BlockSpec((BN, 128), lambda i: (i, 0)),
                pl.BlockSpec(wx.shape, lambda i: (0, 0)),
            ],
            out_specs=pl.BlockSpec((BN, 32), lambda i: (i, 0)),
            out_shape=jax.ShapeDtypeStruct((n, 32), jnp.float32),
        )(x, wx)
    return pl.pallas_call(
        _p_body_u,
        grid=(grid,),
        in_specs=[
            pl.BlockSpec((BN, 128), lambda i: (i, 0)),
            pl.BlockSpec(wx.shape, lambda i: (0, 0)),
            pl.BlockSpec((1, 1, BN), lambda i: (i, 0, 0)),
            pl.BlockSpec(uproj.shape, lambda i: (0, 0)),
        ],
        out_specs=pl.BlockSpec((BN, 32), lambda i: (i, 0)),
        out_shape=jax.ShapeDtypeStruct((n, 32), jnp.float32),
    )(x, wx, batch3, uproj)


def _node_body(x_ref, m_ref, b3_ref, wx, wm, nbr, x_out, xg_out, *, has_u,
               uproj_ref=None):
    m = m_ref[...]
    z = (jnp.dot(x_ref[...], wx[...], precision=_PREC, preferred_element_type=jnp.float32)
         + _dg(m, wm[...], 1, 0) + nbr[...])
    oh = _onehot_t(b3_ref[0])
    if has_u:
        z = z + _dg(oh, uproj_ref[...], 0, 0)
    xn = _relu(z)
    x_out[...] = xn

    @pl.when(pl.program_id(0) == 0)
    def _():
        xg_out[...] = jnp.zeros_like(xg_out)

    xg_out[...] += _dg(oh, xn, 1, 0)


def _node_mat(x, msg_p, batch3, wx, wm, nbr, uprojn):
    n = x.shape[0]
    grid = n // BN
    has_u = uprojn is not None

    if has_u:
        def body(x_ref, m_ref, b3_ref, wx_r, wm_r, nb_r, up_r, x_out, xg_out):
            _node_body(x_ref, m_ref, b3_ref, wx_r, wm_r, nb_r, x_out, xg_out,
                       has_u=True, uproj_ref=up_r)
        extra_specs = [pl.BlockSpec(uprojn.shape, lambda i: (0, 0))]
        args = (x, msg_p, batch3, wx, wm, nbr, uprojn)
    else:
        def body(x_ref, m_ref, b3_ref, wx_r, wm_r, nb_r, x_out, xg_out):
            _node_body(x_ref, m_ref, b3_ref, wx_r, wm_r, nb_r, x_out, xg_out,
                       has_u=False)
        extra_specs = []
        args = (x, msg_p, batch3, wx, wm, nbr)

    return pl.pallas_call(
        body,
        grid=(grid,),
        in_specs=[
            pl.BlockSpec((BN, 128), lambda i: (i, 0)),
            pl.BlockSpec((BN, 32), lambda i: (i, 0)),
            pl.BlockSpec((1, 1, BN), lambda i: (i, 0, 0)),
            pl.BlockSpec(wx.shape, lambda i: (0, 0)),
            pl.BlockSpec(wm.shape, lambda i: (0, 0)),
            pl.BlockSpec(nbr.shape, lambda i: (0, 0)),
        ] + extra_specs,
        out_specs=[
            pl.BlockSpec((BN, 128), lambda i: (i, 0)),
            pl.BlockSpec((G, 128), lambda i: (0, 0)),
        ],
        out_shape=[
            jax.ShapeDtypeStruct((n, 128), jnp.float32),
            jax.ShapeDtypeStruct((G, 128), jnp.float32),
        ],
    )(*args)


def _global_mat(eg_p, xg, u_prev, we, wxg, wu, gbr,
                eproj_w=None, nproj_w=None, ro_w=None, ro_br=None):
    has_u = u_prev is not None
    has_proj = eproj_w is not None

    in_arrays = [eg_p, xg, we, wxg, gbr]
    if has_u:
        in_arrays += [u_prev, wu]
    out_shape = [jax.ShapeDtypeStruct((G, 32), jnp.float32)]
    if has_proj:
        in_arrays += [eproj_w, nproj_w]
        out_shape += [jax.ShapeDtypeStruct((G, 32), jnp.float32),
                      jax.ShapeDtypeStruct((G, 128), jnp.float32)]
    else:
        in_arrays += [ro_w, ro_br]
        out_shape += [jax.ShapeDtypeStruct((1, G), jnp.float32)]

    def body(*refs):
        k = 5
        egp_r, xg_r, we_r, wxg_r, gb_r = refs[:k]
        if has_u:
            u_r, wu_r = refs[k:k + 2]
            k += 2
        a_r, b_r = refs[k:k + 2]
        k += 2
        outs = refs[k:]
        eg = egp_r[0] + egp_r[1]
        z = (_dg(eg, we_r[...], 1, 0) + _dg(xg_r[...], wxg_r[...], 1, 0)
             + gb_r[...])
        if has_u:
            z = z + _dg(u_r[...], wu_r[...], 1, 0)
        uu = _relu(z)
        outs[0][...] = uu
        if has_proj:
            outs[1][...] = _dg(uu, a_r[...], 1, 0)
            outs[2][...] = _dg(uu, b_r[...], 1, 0)
        else:
            outs[1][...] = _dg(a_r[...], uu, 0, 1) + b_r[...]

    full = lambda a: pl.BlockSpec(a.shape, None)
    return pl.pallas_call(
        body,
        in_specs=[full(a) for a in in_arrays],
        out_specs=[pl.BlockSpec(s.shape, None) for s in out_shape],
        out_shape=out_shape,
    )(*in_arrays)


_NW = 32
_EPW = 25000
_CH = 1000
_NCH = _EPW // _CH
_NPAD = 50048
_NPT = _NPAD // 16


def _sc_mesh():
    return plsc.VectorSubcoreMesh(core_axis_name="c", subcore_axis_name="s")


def _compute_eb(src, batch):
    n = batch.shape[0]
    e_cnt = src.shape[0]

    @functools.partial(
        pl.kernel,
        mesh=_sc_mesh(),
        compiler_params=pltpu.CompilerParams(needs_layout_passes=False, use_tc_tiling_on_sc=False),
        out_type=jax.ShapeDtypeStruct((e_cnt,), jnp.int32),
        scratch_types=[
            pltpu.VMEM((n,), jnp.int32),
            pltpu.VMEM((_EPW,), jnp.int32),
            pltpu.VMEM((_EPW,), jnp.int32),
        ],
    )
    def k(src_hbm, batch_hbm, eb_out, batch_v, src_v, eb_v):
        wid = lax.axis_index("s") * 2 + lax.axis_index("c")
        base = wid * _EPW
        pltpu.sync_copy(batch_hbm, batch_v)
        pltpu.sync_copy(src_hbm.at[pl.ds(base, _EPW)], src_v)

        def body(j, _):
            off = jnp.minimum(j * 16, _EPW - 16)
            idx = src_v[pl.ds(off, 16)]
            eb_v[pl.ds(off, 16)] = plsc.load_gather(batch_v, [idx])
            return 0

        lax.fori_loop(0, (_EPW + 15) // 16, body, 0)
        pltpu.sync_copy(eb_v, eb_out.at[pl.ds(base, _EPW)])

    return k(src, batch)


def _edge_pass(p, q, src, eb):
    e_cnt = q.shape[0]

    @functools.partial(
        pl.kernel,
        mesh=_sc_mesh(),
        compiler_params=pltpu.CompilerParams(needs_layout_passes=False,
                                             use_tc_tiling_on_sc=False),
        out_type=[
            jax.ShapeDtypeStruct((e_cnt, 32), jnp.float32),
            jax.ShapeDtypeStruct((2, G, 32), jnp.float32),
        ],
        scratch_types=[
            pltpu.VMEM((_CH,), jnp.int32),
            pltpu.VMEM((_CH,), jnp.int32),
            pltpu.VMEM((_CH,), jnp.int32),
            pltpu.VMEM((_CH, 32), jnp.float32),
            pltpu.VMEM((_CH, 32), jnp.float32),
            pltpu.VMEM((_CH, 32), jnp.float32),
            pltpu.VMEM((16, 32), jnp.float32),
            pltpu.VMEM_SHARED((G, 32), jnp.float32),
            pltpu.SemaphoreType.DMA,
            pltpu.SemaphoreType.DMA,
        ],
    )
    def ka(p_hbm, q_hbm, src_hbm, eb_hbm, e_out, eg_out,
           src0, src1, eb_v, pg0, pg1, q_v, zb_v, eg_sh, sem0, sem1):
        c = lax.axis_index("c")
        s = lax.axis_index("s")
        wid = s * 2 + c
        zero16 = jnp.zeros((16,), jnp.float32)

        def zfill(i, _):
            zb_v[i, pl.ds(0, 16)] = zero16
            zb_v[i, pl.ds(16, 16)] = zero16
            return 0

        lax.fori_loop(0, 16, zfill, 0)
        pltpu.sync_copy(zb_v, eg_sh.at[pl.ds(s * 16, 16)])
        plsc.subcore_barrier()

        base = wid * _EPW
        sets = ((src0, pg0, sem0), (src1, pg1, sem1))

        def issue(g, st):
            src_v, pg_v, sem = st
            off = base + g * _CH
            pltpu.sync_copy(src_hbm.at[pl.ds(off, _CH)], src_v)
            pltpu.async_copy(p_hbm.at[src_v], pg_v, sem)

        def proc(g, st):
            src_v, pg_v, sem = st
            off = base + g * _CH
            pltpu.make_async_copy(p_hbm.at[src_v], pg_v, sem).wait()
            pltpu.sync_copy(q_hbm.at[pl.ds(off, _CH)], q_v)

            def rows(rr, _):
                for u in range(4):
                    r = rr * 4 + u
                    a0 = pg_v[r, pl.ds(0, 16)] + q_v[r, pl.ds(0, 16)]
                    a1 = pg_v[r, pl.ds(16, 16)] + q_v[r, pl.ds(16, 16)]
                    q_v[r, pl.ds(0, 16)] = jnp.maximum(a0, 0.0)
                    q_v[r, pl.ds(16, 16)] = jnp.maximum(a1, 0.0)
                return 0

            lax.fori_loop(0, _CH // 4, rows, 0)
            pltpu.sync_copy(q_v, e_out.at[pl.ds(off, _CH)])
            pltpu.sync_copy(eb_hbm.at[pl.ds(off, _CH)], eb_v)
            pltpu.sync_copy(q_v, eg_sh.at[eb_v], add=True)

        issue(0, sets[0])

        def pair(i, _):
            g = 2 * i
            issue(g + 1, sets[1])
            proc(g, sets[0])
            issue(g + 2, sets[0])
            proc(g + 1, sets[1])
            return 0

        lax.fori_loop(0, (_NCH - 1) // 2, pair, 0)
        proc(_NCH - 1, sets[0])

        plsc.subcore_barrier()
        pltpu.sync_copy(eg_sh.at[pl.ds(s * 16, 16)],
                        eg_out.at[c, pl.ds(s * 16, 16)])

    return ka(p, q, src, eb)


_QR = _NPAD // 4
_QPT = _QR // 16
_DUMP = 2048
_CHB = 1000
_EPT_B = 800000 // 16
_NCHB = _EPT_B // _CHB


def _msg_pass(dst, e_new):
    @functools.partial(
        pl.kernel,
        mesh=_sc_mesh(),
        compiler_params=pltpu.CompilerParams(needs_layout_passes=False,
                                             use_tc_tiling_on_sc=False),
        out_type=jax.ShapeDtypeStruct((_NPAD, 32), jnp.float32),
        scratch_types=[
            pltpu.VMEM((_CHB,), jnp.int32),
            pltpu.VMEM((_CHB,), jnp.int32),
            pltpu.VMEM((_CHB,), jnp.int32),
            pltpu.VMEM((_CHB,), jnp.int32),
            pltpu.VMEM((_CHB, 32), jnp.float32),
            pltpu.VMEM((_CHB, 32), jnp.float32),
            pltpu.VMEM((_QPT, 32), jnp.float32),
            pltpu.VMEM_SHARED((_QR + _DUMP, 32), jnp.float32),
            pltpu.SemaphoreType.DMA,
            pltpu.SemaphoreType.DMA,
            pltpu.SemaphoreType.DMA,
            pltpu.SemaphoreType.DMA,
        ],
    )
    def kb(dst_hbm, e_hbm, msg_out, d0, d1, iq0, iq1, r0, r1, zb_v, acc,
           semd0, seme0, semd1, seme1):
        c = lax.axis_index("c")
        s = lax.axis_index("s")
        zero16 = jnp.zeros((16,), jnp.float32)

        def zfill(i, _):
            zb_v[i, pl.ds(0, 16)] = zero16
            zb_v[i, pl.ds(16, 16)] = zero16
            return 0

        lax.fori_loop(0, _QPT, zfill, 0)

        sets = ((d0, iq0, r0, semd0, seme0), (d1, iq1, r1, semd1, seme1))

        def issue(g, st):
            d_v, iq_v, rows_v, semd, seme = st
            off = s * _EPT_B + g * _CHB
            pltpu.async_copy(dst_hbm.at[pl.ds(off, _CHB)], d_v, semd)
            pltpu.async_copy(e_hbm.at[pl.ds(off, _CHB)], rows_v, seme)

        for qq in range(2):
            qlo = (2 * c + qq) * _QR
            pltpu.sync_copy(zb_v, acc.at[pl.ds(s * _QPT, _QPT)])
            for t in range(_DUMP // _QPT + 1):
                lo = jnp.minimum(_QR + t * _QPT, _QR + _DUMP - _QPT)
                pltpu.sync_copy(zb_v, acc.at[pl.ds(lo, _QPT)])
            plsc.subcore_barrier()

            def proc(g, st):
                d_v, iq_v, rows_v, semd, seme = st
                off = s * _EPT_B + g * _CHB
                pltpu.make_async_copy(dst_hbm.at[pl.ds(off, _CHB)],
                                      d_v, semd).wait()
                pltpu.make_async_copy(e_hbm.at[pl.ds(off, _CHB)],
                                      rows_v, seme).wait()

                def vec(ii, _):
                    for u in range(4):
                        i = ii * 4 + u
                        d = d_v[pl.ds(i * 16, 16)]
                        dq = d - qlo
                        m = (dq >= 0) & (dq < _QR)
                        sp = _QR + ((g * _CHB + i * 16
                                     + lax.broadcasted_iota(jnp.int32, (16,), 0))
                                    & (_DUMP - 1))
                        iq_v[pl.ds(i * 16, 16)] = jnp.where(m, dq, sp)
                    return 0

                lax.fori_loop(0, 15, vec, 0)
                for off_t in (960, 976, 984):
                    d = d_v[pl.ds(off_t, 16)]
                    dq = d - qlo
                    m = (dq >= 0) & (dq < _QR)
                    sp = _QR + ((g * _CHB + off_t
                                 + lax.broadcasted_iota(jnp.int32, (16,), 0))
                                & (_DUMP - 1))
                    iq_v[pl.ds(off_t, 16)] = jnp.where(m, dq, sp)
                pltpu.sync_copy(rows_v, acc.at[iq_v], add=True)

            issue(0, sets[0])

            def pair(i, _):
                g = 2 * i
                issue(g + 1, sets[1])
                proc(g, sets[0])
                issue(g + 2, sets[0])
                proc(g + 1, sets[1])
                return 0

            lax.fori_loop(0, (_NCHB - 2) // 2, pair, 0)
            issue(_NCHB - 1, sets[1])
            proc(_NCHB - 2, sets[0])
            proc(_NCHB - 1, sets[1])

            plsc.subcore_barrier()
            pltpu.sync_copy(acc.at[pl.ds(s * _QPT, _QPT)],
                            msg_out.at[pl.ds(qlo + s * _QPT, _QPT)])
            plsc.subcore_barrier()

    return kb(dst, e_new)


def kernel(x, edge_index, edge_attr, batch,
           en_W1, en_b1, en_W2, en_b2,
           ee_W1, ee_b1, ee_W2, ee_b2,
           l1_eW, l1_eb, l1_nW, l1_nb, l1_gW, l1_gb,
           l2_eW, l2_eb, l2_nW, l2_nb, l2_gW, l2_gb,
           l3_eW, l3_eb, l3_nW, l3_nb, l3_gW, l3_gb,
           ro_W, ro_b):
    n = x.shape[0]
    e_cnt = edge_index.shape[1]
    src = edge_index[0]
    dst = edge_index[1]
    batch3 = batch.reshape(n // BN, 1, BN)

    row = lambda v: v.reshape(1, -1)
    col = lambda v: v.reshape(-1, 1)

    x0 = _enc_node(x, en_W1, row(en_b1), en_W2, row(en_b2))
    q1 = _enc_edge_q1(edge_attr.reshape(1, e_cnt), col(ee_W1[0]),
                      col(ee_b1), ee_W2.T, col(ee_b2),
                      l1_eW[128:160], row(l1_eb))

    eb_arr = _compute_eb(src, batch)

    layers = [
        (l1_eW, l1_eb, l1_nW, l1_nb, l1_gW, l1_gb, False),
        (l2_eW, l2_eb, l2_nW, l2_nb, l2_gW, l2_gb, True),
        (l3_eW, l3_eb, l3_nW, l3_nb, l3_gW, l3_gb, True),
    ]

    xc = x0
    u = None
    uproj_e = None
    uproj_n = None
    q = q1
    e_arr = None
    res = None

    for li, (eW, ebias, nW, nb, gW, gb, has_u) in enumerate(layers):
        if li > 0:
            q = _q_mat(e_arr, eW[128:160], row(ebias))
        p = _p_mat(xc, eW[:128], batch3, uproj_e if has_u else None)

        e_arr, eg_p = _edge_pass(p, q, src, eb_arr)
        msg = _msg_pass(dst, e_arr)

        xc, xg = _node_mat(xc, msg, batch3, nW[:128], nW[128:160],
                           row(nb), uproj_n if has_u else None)

        is_last = li == 2
        if not is_last:
            nxt_eW = layers[li + 1][0]
            nxt_nW = layers[li + 1][2]
            outs = _global_mat(eg_p, xg, u, gW[:32], gW[32:160],
                               gW[160:192] if has_u else None, row(gb),
                               eproj_w=nxt_eW[160:192], nproj_w=nxt_nW[160:192])
            u, uproj_e, uproj_n = outs
        else:
            outs = _global_mat(eg_p, xg, u, gW[:32], gW[32:160],
                               gW[160:192], row(gb),
                               ro_w=ro_W, ro_br=row(ro_b))
            res = outs[1]

    return res.reshape(-1)


# speedup vs baseline: 4.9507x; 1.0224x over previous
import functools

import jax
import jax.numpy as jnp
from jax import lax
from jax.experimental import pallas as pl
from jax.experimental.pallas import tpu as pltpu
from jax.experimental.pallas import tpu_sc as plsc

G = 256

BN = 2000
BE = 8000
BA = 6400


_PREC = lax.Precision.HIGHEST


def _dg(a, b, ca, cb):
    return lax.dot_general(a, b, (((ca,), (cb,)), ((), ())),
                           preferred_element_type=jnp.float32,
                           precision=_PREC)


def _relu(v):
    return jnp.maximum(v, 0.0)


def _enc_node_body(x_ref, w1, b1, w2, b2, o_ref):
    h = _relu(jnp.dot(x_ref[...], w1[...], precision=_PREC,
                      preferred_element_type=jnp.float32) + b1[...])
    o_ref[...] = _relu(jnp.dot(h, w2[...], precision=_PREC,
                               preferred_element_type=jnp.float32) + b2[...])


def _enc_node(x, w1, b1r, w2, b2r):
    n = x.shape[0]
    grid = n // BN
    return pl.pallas_call(
        _enc_node_body,
        grid=(grid,),
        in_specs=[
            pl.BlockSpec((BN, x.shape[1]), lambda i: (i, 0)),
            pl.BlockSpec(w1.shape, lambda i: (0, 0)),
            pl.BlockSpec(b1r.shape, lambda i: (0, 0)),
            pl.BlockSpec(w2.shape, lambda i: (0, 0)),
            pl.BlockSpec(b2r.shape, lambda i: (0, 0)),
        ],
        out_specs=pl.BlockSpec((BN, w2.shape[1]), lambda i: (i, 0)),
        out_shape=jax.ShapeDtypeStruct((n, w2.shape[1]), jnp.float32),
    )(x, w1, b1r, w2, b2r)


def _enc_edge_body(a_ref, w1c, b1c, w2t, b2c, we, ber, o_ref):
    a = a_ref[...]
    h1 = _relu(w1c[...] * a + b1c[...])
    e0 = _relu(_dg(w2t[...], h1, 1, 0) + b2c[...])
    o_ref[...] = _dg(e0, we[...], 0, 0) + ber[...]


def _enc_edge_q1(a_row, w1c, b1c, w2t, b2c, we, ber):
    e = a_row.shape[1]
    grid = e // BA
    return pl.pallas_call(
        _enc_edge_body,
        grid=(grid,),
        in_specs=[
            pl.BlockSpec((1, BA), lambda i: (0, i)),
            pl.BlockSpec(w1c.shape, lambda i: (0, 0)),
            pl.BlockSpec(b1c.shape, lambda i: (0, 0)),
            pl.BlockSpec(w2t.shape, lambda i: (0, 0)),
            pl.BlockSpec(b2c.shape, lambda i: (0, 0)),
            pl.BlockSpec(we.shape, lambda i: (0, 0)),
            pl.BlockSpec(ber.shape, lambda i: (0, 0)),
        ],
        out_specs=pl.BlockSpec((BA, 32), lambda i: (i, 0)),
        out_shape=jax.ShapeDtypeStruct((e, 32), jnp.float32),
    )(a_row, w1c, b1c, w2t, b2c, we, ber)


def _q_body(e_ref, we, ber, o_ref):
    o_ref[...] = jnp.dot(e_ref[...], we[...], precision=_PREC,
                         preferred_element_type=jnp.float32) + ber[...]


def _q_mat(e, we, ber):
    n = e.shape[0]
    grid = n // BE
    return pl.pallas_call(
        _q_body,
        grid=(grid,),
        in_specs=[
            pl.BlockSpec((BE, 32), lambda i: (i, 0)),
            pl.BlockSpec(we.shape, lambda i: (0, 0)),
            pl.BlockSpec(ber.shape, lambda i: (0, 0)),
        ],
        out_specs=pl.BlockSpec((BE, 32), lambda i: (i, 0)),
        out_shape=jax.ShapeDtypeStruct((n, 32), jnp.float32),
    )(e, we, ber)


def _onehot_t(batch_row):
    segs = lax.broadcasted_iota(jnp.int32, (G, batch_row.shape[1]), 0)
    return jnp.where(segs == batch_row, 1.0, 0.0).astype(jnp.float32)


def _p_body_u(x_ref, wx, b3_ref, uproj, o_ref):
    oh = _onehot_t(b3_ref[0])
    ut = _dg(oh, uproj[...], 0, 0)
    o_ref[...] = jnp.dot(x_ref[...], wx[...], precision=_PREC,
                         preferred_element_type=jnp.float32) + ut


def _p_body(x_ref, wx, o_ref):
    o_ref[...] = jnp.dot(x_ref[...], wx[...], precision=_PREC,
                         preferred_element_type=jnp.float32)


def _p_mat(x, wx, batch3, uproj):
    n = x.shape[0]
    grid = n // BN
    if uproj is None:
        return pl.pallas_call(
            _p_body,
            grid=(grid,),
            in_specs=[
                pl.BlockSpec((BN, 128), lambda i: (i, 0)),
                pl.BlockSpec(wx.shape, lambda i: (0, 0)),
            ],
            out_specs=pl.BlockSpec((BN, 32), lambda i: (i, 0)),
            out_shape=jax.ShapeDtypeStruct((n, 32), jnp.float32),
        )(x, wx)
    return pl.pallas_call(
        _p_body_u,
        grid=(grid,),
        in_specs=[
            pl.BlockSpec((BN, 128), lambda i: (i, 0)),
            pl.BlockSpec(wx.shape, lambda i: (0, 0)),
            pl.BlockSpec((1, 1, BN), lambda i: (i, 0, 0)),
            pl.BlockSpec(uproj.shape, lambda i: (0, 0)),
        ],
        out_specs=pl.BlockSpec((BN, 32), lambda i: (i, 0)),
        out_shape=jax.ShapeDtypeStruct((n, 32), jnp.float32),
    )(x, wx, batch3, uproj)


def _node_body(x_ref, m_ref, b3_ref, wx, wm, nbr, x_out, xg_out, *, has_u,
               uproj_ref=None):
    m = m_ref[...]
    z = (jnp.dot(x_ref[...], wx[...], precision=_PREC, preferred_element_type=jnp.float32)
         + _dg(m, wm[...], 1, 0) + nbr[...])
    oh = _onehot_t(b3_ref[0])
    if has_u:
        z = z + _dg(oh, uproj_ref[...], 0, 0)
    xn = _relu(z)
    x_out[...] = xn

    @pl.when(pl.program_id(0) == 0)
    def _():
        xg_out[...] = jnp.zeros_like(xg_out)

    xg_out[...] += _dg(oh, xn, 1, 0)


def _node_mat(x, msg_p, batch3, wx, wm, nbr, uprojn):
    n = x.shape[0]
    grid = n // BN
    has_u = uprojn is not None

    if has_u:
        def body(x_ref, m_ref, b3_ref, wx_r, wm_r, nb_r, up_r, x_out, xg_out):
            _node_body(x_ref, m_ref, b3_ref, wx_r, wm_r, nb_r, x_out, xg_out,
                       has_u=True, uproj_ref=up_r)
        extra_specs = [pl.BlockSpec(uprojn.shape, lambda i: (0, 0))]
        args = (x, msg_p, batch3, wx, wm, nbr, uprojn)
    else:
        def body(x_ref, m_ref, b3_ref, wx_r, wm_r, nb_r, x_out, xg_out):
            _node_body(x_ref, m_ref, b3_ref, wx_r, wm_r, nb_r, x_out, xg_out,
                       has_u=False)
        extra_specs = []
        args = (x, msg_p, batch3, wx, wm, nbr)

    return pl.pallas_call(
        body,
        grid=(grid,),
        in_specs=[
            pl.BlockSpec((BN, 128), lambda i: (i, 0)),
            pl.BlockSpec((BN, 32), lambda i: (i, 0)),
            pl.BlockSpec((1, 1, BN), lambda i: (i, 0, 0)),
            pl.BlockSpec(wx.shape, lambda i: (0, 0)),
            pl.BlockSpec(wm.shape, lambda i: (0, 0)),
            pl.BlockSpec(nbr.shape, lambda i: (0, 0)),
        ] + extra_specs,
        out_specs=[
            pl.BlockSpec((BN, 128), lambda i: (i, 0)),
            pl.BlockSpec((G, 128), lambda i: (0, 0)),
        ],
        out_shape=[
            jax.ShapeDtypeStruct((n, 128), jnp.float32),
            jax.ShapeDtypeStruct((G, 128), jnp.float32),
        ],
    )(*args)


def _global_mat(eg_p, xg, u_prev, we, wxg, wu, gbr,
                eproj_w=None, nproj_w=None, ro_w=None, ro_br=None):
    has_u = u_prev is not None
    has_proj = eproj_w is not None

    in_arrays = [eg_p, xg, we, wxg, gbr]
    if has_u:
        in_arrays += [u_prev, wu]
    out_shape = [jax.ShapeDtypeStruct((G, 32), jnp.float32)]
    if has_proj:
        in_arrays += [eproj_w, nproj_w]
        out_shape += [jax.ShapeDtypeStruct((G, 32), jnp.float32),
                      jax.ShapeDtypeStruct((G, 128), jnp.float32)]
    else:
        in_arrays += [ro_w, ro_br]
        out_shape += [jax.ShapeDtypeStruct((1, G), jnp.float32)]

    def body(*refs):
        k = 5
        egp_r, xg_r, we_r, wxg_r, gb_r = refs[:k]
        if has_u:
            u_r, wu_r = refs[k:k + 2]
            k += 2
        a_r, b_r = refs[k:k + 2]
        k += 2
        outs = refs[k:]
        eg = egp_r[0] + egp_r[1]
        z = (_dg(eg, we_r[...], 1, 0) + _dg(xg_r[...], wxg_r[...], 1, 0)
             + gb_r[...])
        if has_u:
            z = z + _dg(u_r[...], wu_r[...], 1, 0)
        uu = _relu(z)
        outs[0][...] = uu
        if has_proj:
            outs[1][...] = _dg(uu, a_r[...], 1, 0)
            outs[2][...] = _dg(uu, b_r[...], 1, 0)
        else:
            outs[1][...] = _dg(a_r[...], uu, 0, 1) + b_r[...]

    full = lambda a: pl.BlockSpec(a.shape, None)
    return pl.pallas_call(
        body,
        in_specs=[full(a) for a in in_arrays],
        out_specs=[pl.BlockSpec(s.shape, None) for s in out_shape],
        out_shape=out_shape,
    )(*in_arrays)


_NW = 32
_EPW = 25000
_CH = 1000
_NCH = _EPW // _CH
_NPAD = 50048
_NPT = _NPAD // 16


def _sc_mesh():
    return plsc.VectorSubcoreMesh(core_axis_name="c", subcore_axis_name="s")


def _compute_eb(src, batch):
    n = batch.shape[0]
    e_cnt = src.shape[0]

    @functools.partial(
        pl.kernel,
        mesh=_sc_mesh(),
        compiler_params=pltpu.CompilerParams(needs_layout_passes=False, use_tc_tiling_on_sc=False),
        out_type=jax.ShapeDtypeStruct((e_cnt,), jnp.int32),
        scratch_types=[
            pltpu.VMEM((n,), jnp.int32),
            pltpu.VMEM((_EPW,), jnp.int32),
            pltpu.VMEM((_EPW,), jnp.int32),
        ],
    )
    def k(src_hbm, batch_hbm, eb_out, batch_v, src_v, eb_v):
        wid = lax.axis_index("s") * 2 + lax.axis_index("c")
        base = wid * _EPW
        pltpu.sync_copy(batch_hbm, batch_v)
        pltpu.sync_copy(src_hbm.at[pl.ds(base, _EPW)], src_v)

        def body(j, _):
            off = jnp.minimum(j * 16, _EPW - 16)
            idx = src_v[pl.ds(off, 16)]
            eb_v[pl.ds(off, 16)] = plsc.load_gather(batch_v, [idx])
            return 0

        lax.fori_loop(0, (_EPW + 15) // 16, body, 0)
        pltpu.sync_copy(eb_v, eb_out.at[pl.ds(base, _EPW)])

    return k(src, batch)


def _edge_pass(p, q, src, eb):
    e_cnt = q.shape[0]

    @functools.partial(
        pl.kernel,
        mesh=_sc_mesh(),
        compiler_params=pltpu.CompilerParams(needs_layout_passes=False,
                                             use_tc_tiling_on_sc=False),
        out_type=[
            jax.ShapeDtypeStruct((e_cnt, 32), jnp.float32),
            jax.ShapeDtypeStruct((2, G, 32), jnp.float32),
        ],
        scratch_types=[
            pltpu.VMEM((_CH,), jnp.int32),
            pltpu.VMEM((_CH,), jnp.int32),
            pltpu.VMEM((_CH,), jnp.int32),
            pltpu.VMEM((_CH, 32), jnp.float32),
            pltpu.VMEM((_CH, 32), jnp.float32),
            pltpu.VMEM((_CH, 32), jnp.float32),
            pltpu.VMEM((16, 32), jnp.float32),
            pltpu.VMEM_SHARED((G, 32), jnp.float32),
            pltpu.SemaphoreType.DMA,
            pltpu.SemaphoreType.DMA,
            pltpu.SemaphoreType.DMA,
            pltpu.SemaphoreType.DMA,
        ],
    )
    def ka(p_hbm, q_hbm, src_hbm, eb_hbm, e_out, eg_out,
           src0, src1, eb_v, pg0, pg1, q_v, zb_v, eg_sh, sem0, sem1,
           sem_eb, sem_eo):
        c = lax.axis_index("c")
        s = lax.axis_index("s")
        wid = s * 2 + c
        zero16 = jnp.zeros((16,), jnp.float32)

        def zfill(i, _):
            zb_v[i, pl.ds(0, 16)] = zero16
            zb_v[i, pl.ds(16, 16)] = zero16
            return 0

        lax.fori_loop(0, 16, zfill, 0)
        pltpu.sync_copy(zb_v, eg_sh.at[pl.ds(s * 16, 16)])
        plsc.subcore_barrier()

        base = wid * _EPW
        sets = ((src0, pg0, sem0), (src1, pg1, sem1))

        def issue(g, st):
            src_v, pg_v, sem = st
            off = base + g * _CH
            pltpu.sync_copy(src_hbm.at[pl.ds(off, _CH)], src_v)
            pltpu.async_copy(p_hbm.at[src_v], pg_v, sem)

        def proc(g, st):
            src_v, pg_v, sem = st
            off = base + g * _CH
            pltpu.async_copy(eb_hbm.at[pl.ds(off, _CH)], eb_v, sem_eb)
            pltpu.make_async_copy(p_hbm.at[src_v], pg_v, sem).wait()
            pltpu.sync_copy(q_hbm.at[pl.ds(off, _CH)], q_v)

            def rows(rr, _):
                for u in range(4):
                    r = rr * 4 + u
                    a0 = pg_v[r, pl.ds(0, 16)] + q_v[r, pl.ds(0, 16)]
                    a1 = pg_v[r, pl.ds(16, 16)] + q_v[r, pl.ds(16, 16)]
                    q_v[r, pl.ds(0, 16)] = jnp.maximum(a0, 0.0)
                    q_v[r, pl.ds(16, 16)] = jnp.maximum(a1, 0.0)
                return 0

            lax.fori_loop(0, _CH // 4, rows, 0)
            pltpu.async_copy(q_v, e_out.at[pl.ds(off, _CH)], sem_eo)
            pltpu.make_async_copy(eb_hbm.at[pl.ds(off, _CH)], eb_v,
                                  sem_eb).wait()
            pltpu.sync_copy(q_v, eg_sh.at[eb_v], add=True)
            pltpu.make_async_copy(q_v, e_out.at[pl.ds(off, _CH)],
                                  sem_eo).wait()

        issue(0, sets[0])

        def pair(i, _):
            g = 2 * i
            issue(g + 1, sets[1])
            proc(g, sets[0])
            issue(g + 2, sets[0])
            proc(g + 1, sets[1])
            return 0

        lax.fori_loop(0, (_NCH - 1) // 2, pair, 0)
        proc(_NCH - 1, sets[0])

        plsc.subcore_barrier()
        pltpu.sync_copy(eg_sh.at[pl.ds(s * 16, 16)],
                        eg_out.at[c, pl.ds(s * 16, 16)])

    return ka(p, q, src, eb)


_QR = _NPAD // 4
_QPT = _QR // 16
_DUMP = 2048
_CHB = 1000
_EPT_B = 800000 // 16
_NCHB = _EPT_B // _CHB


def _msg_pass(dst, e_new):
    @functools.partial(
        pl.kernel,
        mesh=_sc_mesh(),
        compiler_params=pltpu.CompilerParams(needs_layout_passes=False,
                                             use_tc_tiling_on_sc=False),
        out_type=jax.ShapeDtypeStruct((_NPAD, 32), jnp.float32),
        scratch_types=[
            pltpu.VMEM((_CHB,), jnp.int32),
            pltpu.VMEM((_CHB,), jnp.int32),
            pltpu.VMEM((_CHB,), jnp.int32),
            pltpu.VMEM((_CHB,), jnp.int32),
            pltpu.VMEM((_CHB, 32), jnp.float32),
            pltpu.VMEM((_CHB, 32), jnp.float32),
            pltpu.VMEM((_QPT, 32), jnp.float32),
            pltpu.VMEM_SHARED((_QR + _DUMP, 32), jnp.float32),
            pltpu.SemaphoreType.DMA,
            pltpu.SemaphoreType.DMA,
            pltpu.SemaphoreType.DMA,
            pltpu.SemaphoreType.DMA,
        ],
    )
    def kb(dst_hbm, e_hbm, msg_out, d0, d1, iq0, iq1, r0, r1, zb_v, acc,
           semd0, seme0, semd1, seme1):
        c = lax.axis_index("c")
        s = lax.axis_index("s")
        zero16 = jnp.zeros((16,), jnp.float32)

        def zfill(i, _):
            zb_v[i, pl.ds(0, 16)] = zero16
            zb_v[i, pl.ds(16, 16)] = zero16
            return 0

        lax.fori_loop(0, _QPT, zfill, 0)

        sets = ((d0, iq0, r0, semd0, seme0), (d1, iq1, r1, semd1, seme1))

        def issue(g, st):
            d_v, iq_v, rows_v, semd, seme = st
            off = s * _EPT_B + g * _CHB
            pltpu.async_copy(dst_hbm.at[pl.ds(off, _CHB)], d_v, semd)
            pltpu.async_copy(e_hbm.at[pl.ds(off, _CHB)], rows_v, seme)

        for qq in range(2):
            qlo = (2 * c + qq) * _QR
            pltpu.sync_copy(zb_v, acc.at[pl.ds(s * _QPT, _QPT)])
            for t in range(_DUMP // _QPT + 1):
                lo = jnp.minimum(_QR + t * _QPT, _QR + _DUMP - _QPT)
                pltpu.sync_copy(zb_v, acc.at[pl.ds(lo, _QPT)])
            plsc.subcore_barrier()

            def proc(g, st):
                d_v, iq_v, rows_v, semd, seme = st
                off = s * _EPT_B + g * _CHB
                pltpu.make_async_copy(dst_hbm.at[pl.ds(off, _CHB)],
                                      d_v, semd).wait()

                def vec(ii, _):
                    for u in range(4):
                        i = ii * 4 + u
                        d = d_v[pl.ds(i * 16, 16)]
                        dq = d - qlo
                        m = (dq >= 0) & (dq < _QR)
                        sp = _QR + ((g * _CHB + i * 16
                                     + lax.broadcasted_iota(jnp.int32, (16,), 0))
                                    & (_DUMP - 1))
                        iq_v[pl.ds(i * 16, 16)] = jnp.where(m, dq, sp)
                    return 0

                lax.fori_loop(0, 15, vec, 0)
                for off_t in (960, 976, 984):
                    d = d_v[pl.ds(off_t, 16)]
                    dq = d - qlo
                    m = (dq >= 0) & (dq < _QR)
                    sp = _QR + ((g * _CHB + off_t
                                 + lax.broadcasted_iota(jnp.int32, (16,), 0))
                                & (_DUMP - 1))
                    iq_v[pl.ds(off_t, 16)] = jnp.where(m, dq, sp)
                pltpu.make_async_copy(e_hbm.at[pl.ds(off, _CHB)],
                                      rows_v, seme).wait()
                pltpu.sync_copy(rows_v, acc.at[iq_v], add=True)

            issue(0, sets[0])

            def pair(i, _):
                g = 2 * i
                issue(g + 1, sets[1])
                proc(g, sets[0])
                issue(g + 2, sets[0])
                proc(g + 1, sets[1])
                return 0

            lax.fori_loop(0, (_NCHB - 2) // 2, pair, 0)
            issue(_NCHB - 1, sets[1])
            proc(_NCHB - 2, sets[0])
            proc(_NCHB - 1, sets[1])

            plsc.subcore_barrier()
            pltpu.sync_copy(acc.at[pl.ds(s * _QPT, _QPT)],
                            msg_out.at[pl.ds(qlo + s * _QPT, _QPT)])
            plsc.subcore_barrier()

    return kb(dst, e_new)


def kernel(x, edge_index, edge_attr, batch,
           en_W1, en_b1, en_W2, en_b2,
           ee_W1, ee_b1, ee_W2, ee_b2,
           l1_eW, l1_eb, l1_nW, l1_nb, l1_gW, l1_gb,
           l2_eW, l2_eb, l2_nW, l2_nb, l2_gW, l2_gb,
           l3_eW, l3_eb, l3_nW, l3_nb, l3_gW, l3_gb,
           ro_W, ro_b):
    n = x.shape[0]
    e_cnt = edge_index.shape[1]
    src = edge_index[0]
    dst = edge_index[1]
    batch3 = batch.reshape(n // BN, 1, BN)

    row = lambda v: v.reshape(1, -1)
    col = lambda v: v.reshape(-1, 1)

    x0 = _enc_node(x, en_W1, row(en_b1), en_W2, row(en_b2))
    q1 = _enc_edge_q1(edge_attr.reshape(1, e_cnt), col(ee_W1[0]),
                      col(ee_b1), ee_W2.T, col(ee_b2),
                      l1_eW[128:160], row(l1_eb))

    eb_arr = _compute_eb(src, batch)

    layers = [
        (l1_eW, l1_eb, l1_nW, l1_nb, l1_gW, l1_gb, False),
        (l2_eW, l2_eb, l2_nW, l2_nb, l2_gW, l2_gb, True),
        (l3_eW, l3_eb, l3_nW, l3_nb, l3_gW, l3_gb, True),
    ]

    xc = x0
    u = None
    uproj_e = None
    uproj_n = None
    q = q1
    e_arr = None
    res = None

    for li, (eW, ebias, nW, nb, gW, gb, has_u) in enumerate(layers):
        if li > 0:
            q = _q_mat(e_arr, eW[128:160], row(ebias))
        p = _p_mat(xc, eW[:128], batch3, uproj_e if has_u else None)

        e_arr, eg_p = _edge_pass(p, q, src, eb_arr)
        msg = _msg_pass(dst, e_arr)

        xc, xg = _node_mat(xc, msg, batch3, nW[:128], nW[128:160],
                           row(nb), uproj_n if has_u else None)

        is_last = li == 2
        if not is_last:
            nxt_eW = layers[li + 1][0]
            nxt_nW = layers[li + 1][2]
            outs = _global_mat(eg_p, xg, u, gW[:32], gW[32:160],
                               gW[160:192] if has_u else None, row(gb),
                               eproj_w=nxt_eW[160:192], nproj_w=nxt_nW[160:192])
            u, uproj_e, uproj_n = outs
        else:
            outs = _global_mat(eg_p, xg, u, gW[:32], gW[32:160],
                               gW[160:192], row(gb),
                               ro_w=ro_W, ro_br=row(ro_b))
            res = outs[1]

    return res.reshape(-1)


# speedup vs baseline: 4.9556x; 1.0010x over previous
import functools

import jax
import jax.numpy as jnp
from jax import lax
from jax.experimental import pallas as pl
from jax.experimental.pallas import tpu as pltpu
from jax.experimental.pallas import tpu_sc as plsc

G = 256

BN = 2000
BE = 8000
BA = 6400


_PREC = lax.Precision.HIGHEST


def _dg(a, b, ca, cb):
    return lax.dot_general(a, b, (((ca,), (cb,)), ((), ())),
                           preferred_element_type=jnp.float32,
                           precision=_PREC)


def _relu(v):
    return jnp.maximum(v, 0.0)


def _enc_node_body(x_ref, w1, b1, w2, b2, wx1, o_ref, p_ref):
    h = _relu(jnp.dot(x_ref[...], w1[...], precision=_PREC,
                      preferred_element_type=jnp.float32) + b1[...])
    x0 = _relu(jnp.dot(h, w2[...], precision=_PREC,
                       preferred_element_type=jnp.float32) + b2[...])
    o_ref[...] = x0
    p_ref[...] = jnp.dot(x0, wx1[...], precision=_PREC,
                         preferred_element_type=jnp.float32)


def _enc_node(x, w1, b1r, w2, b2r, wx1):
    n = x.shape[0]
    grid = n // BN
    return pl.pallas_call(
        _enc_node_body,
        grid=(grid,),
        in_specs=[
            pl.BlockSpec((BN, x.shape[1]), lambda i: (i, 0)),
            pl.BlockSpec(w1.shape, lambda i: (0, 0)),
            pl.BlockSpec(b1r.shape, lambda i: (0, 0)),
            pl.BlockSpec(w2.shape, lambda i: (0, 0)),
            pl.BlockSpec(b2r.shape, lambda i: (0, 0)),
            pl.BlockSpec(wx1.shape, lambda i: (0, 0)),
        ],
        out_specs=[
            pl.BlockSpec((BN, w2.shape[1]), lambda i: (i, 0)),
            pl.BlockSpec((BN, 32), lambda i: (i, 0)),
        ],
        out_shape=[
            jax.ShapeDtypeStruct((n, w2.shape[1]), jnp.float32),
            jax.ShapeDtypeStruct((n, 32), jnp.float32),
        ],
    )(x, w1, b1r, w2, b2r, wx1)


def _enc_edge_body(a_ref, w1c, b1c, w2t, b2c, we, ber, o_ref):
    a = a_ref[...]
    h1 = _relu(w1c[...] * a + b1c[...])
    e0 = _relu(_dg(w2t[...], h1, 1, 0) + b2c[...])
    o_ref[...] = _dg(e0, we[...], 0, 0) + ber[...]


def _enc_edge_q1(a_row, w1c, b1c, w2t, b2c, we, ber):
    e = a_row.shape[1]
    grid = e // BA
    return pl.pallas_call(
        _enc_edge_body,
        grid=(grid,),
        in_specs=[
            pl.BlockSpec((1, BA), lambda i: (0, i)),
            pl.BlockSpec(w1c.shape, lambda i: (0, 0)),
            pl.BlockSpec(b1c.shape, lambda i: (0, 0)),
            pl.BlockSpec(w2t.shape, lambda i: (0, 0)),
            pl.BlockSpec(b2c.shape, lambda i: (0, 0)),
            pl.BlockSpec(we.shape, lambda i: (0, 0)),
            pl.BlockSpec(ber.shape, lambda i: (0, 0)),
        ],
        out_specs=pl.BlockSpec((BA, 32), lambda i: (i, 0)),
        out_shape=jax.ShapeDtypeStruct((e, 32), jnp.float32),
    )(a_row, w1c, b1c, w2t, b2c, we, ber)


def _q_body(e_ref, we, ber, o_ref):
    o_ref[...] = jnp.dot(e_ref[...], we[...], precision=_PREC,
                         preferred_element_type=jnp.float32) + ber[...]


def _q_mat(e, we, ber):
    n = e.shape[0]
    grid = n // BE
    return pl.pallas_call(
        _q_body,
        grid=(grid,),
        in_specs=[
            pl.BlockSpec((BE, 32), lambda i: (i, 0)),
            pl.BlockSpec(we.shape, lambda i: (0, 0)),
            pl.BlockSpec(ber.shape, lambda i: (0, 0)),
        ],
        out_specs=pl.BlockSpec((BE, 32), lambda i: (i, 0)),
        out_shape=jax.ShapeDtypeStruct((n, 32), jnp.float32),
    )(e, we, ber)


def _onehot_t(batch_row):
    segs = lax.broadcasted_iota(jnp.int32, (G, batch_row.shape[1]), 0)
    return jnp.where(segs == batch_row, 1.0, 0.0).astype(jnp.float32)


def _padd_body(p_ref, b3_ref, uproj, o_ref):
    oh = _onehot_t(b3_ref[0])
    o_ref[...] = p_ref[...] + _dg(oh, uproj[...], 0, 0)


def _p_add(p_part, batch3, uproj):
    n = p_part.shape[0]
    grid = n // BN
    return pl.pallas_call(
        _padd_body,
        grid=(grid,),
        in_specs=[
            pl.BlockSpec((BN, 32), lambda i: (i, 0)),
            pl.BlockSpec((1, 1, BN), lambda i: (i, 0, 0)),
            pl.BlockSpec(uproj.shape, lambda i: (0, 0)),
        ],
        out_specs=pl.BlockSpec((BN, 32), lambda i: (i, 0)),
        out_shape=jax.ShapeDtypeStruct((n, 32), jnp.float32),
    )(p_part, batch3, uproj)


def _node_body(x_ref, m_ref, b3_ref, wx, wm, nbr, x_out, xg_out, *, has_u,
               uproj_ref=None, wnext_ref=None, p_out=None):
    m = m_ref[...]
    z = (jnp.dot(x_ref[...], wx[...], precision=_PREC, preferred_element_type=jnp.float32)
         + _dg(m, wm[...], 1, 0) + nbr[...])
    oh = _onehot_t(b3_ref[0])
    if has_u:
        z = z + _dg(oh, uproj_ref[...], 0, 0)
    xn = _relu(z)
    x_out[...] = xn
    if p_out is not None:
        p_out[...] = jnp.dot(xn, wnext_ref[...], precision=_PREC,
                             preferred_element_type=jnp.float32)

    @pl.when(pl.program_id(0) == 0)
    def _():
        xg_out[...] = jnp.zeros_like(xg_out)

    xg_out[...] += _dg(oh, xn, 1, 0)


def _node_mat(x, msg_p, batch3, wx, wm, nbr, uprojn, wnext):
    n = x.shape[0]
    grid = n // BN
    has_u = uprojn is not None
    has_p = wnext is not None

    extra_specs = []
    args = [x, msg_p, batch3, wx, wm, nbr]
    if has_u:
        extra_specs.append(pl.BlockSpec(uprojn.shape, lambda i: (0, 0)))
        args.append(uprojn)
    if has_p:
        extra_specs.append(pl.BlockSpec(wnext.shape, lambda i: (0, 0)))
        args.append(wnext)

    def body(*refs):
        k = 6
        x_ref, m_ref, b3_ref, wx_r, wm_r, nb_r = refs[:k]
        up_r = wn_r = None
        if has_u:
            up_r = refs[k]; k += 1
        if has_p:
            wn_r = refs[k]; k += 1
        outs = refs[k:]
        p_o = outs[2] if has_p else None
        _node_body(x_ref, m_ref, b3_ref, wx_r, wm_r, nb_r, outs[0], outs[1],
                   has_u=has_u, uproj_ref=up_r, wnext_ref=wn_r, p_out=p_o)

    out_specs = [
        pl.BlockSpec((BN, 128), lambda i: (i, 0)),
        pl.BlockSpec((G, 128), lambda i: (0, 0)),
    ]
    out_shape = [
        jax.ShapeDtypeStruct((n, 128), jnp.float32),
        jax.ShapeDtypeStruct((G, 128), jnp.float32),
    ]
    if has_p:
        out_specs.append(pl.BlockSpec((BN, 32), lambda i: (i, 0)))
        out_shape.append(jax.ShapeDtypeStruct((n, 32), jnp.float32))

    return pl.pallas_call(
        body,
        grid=(grid,),
        in_specs=[
            pl.BlockSpec((BN, 128), lambda i: (i, 0)),
            pl.BlockSpec((BN, 32), lambda i: (i, 0)),
            pl.BlockSpec((1, 1, BN), lambda i: (i, 0, 0)),
            pl.BlockSpec(wx.shape, lambda i: (0, 0)),
            pl.BlockSpec(wm.shape, lambda i: (0, 0)),
            pl.BlockSpec(nbr.shape, lambda i: (0, 0)),
        ] + extra_specs,
        out_specs=out_specs,
        out_shape=out_shape,
    )(*args)


def _global_mat(eg_p, xg, u_prev, we, wxg, wu, gbr,
                eproj_w=None, nproj_w=None, ro_w=None, ro_br=None):
    has_u = u_prev is not None
    has_proj = eproj_w is not None

    in_arrays = [eg_p, xg, we, wxg, gbr]
    if has_u:
        in_arrays += [u_prev, wu]
    out_shape = [jax.ShapeDtypeStruct((G, 32), jnp.float32)]
    if has_proj:
        in_arrays += [eproj_w, nproj_w]
        out_shape += [jax.ShapeDtypeStruct((G, 32), jnp.float32),
                      jax.ShapeDtypeStruct((G, 128), jnp.float32)]
    else:
        in_arrays += [ro_w, ro_br]
        out_shape += [jax.ShapeDtypeStruct((1, G), jnp.float32)]

    def body(*refs):
        k = 5
        egp_r, xg_r, we_r, wxg_r, gb_r = refs[:k]
        if has_u:
            u_r, wu_r = refs[k:k + 2]
            k += 2
        a_r, b_r = refs[k:k + 2]
        k += 2
        outs = refs[k:]
        eg = egp_r[0] + egp_r[1]
        z = (_dg(eg, we_r[...], 1, 0) + _dg(xg_r[...], wxg_r[...], 1, 0)
             + gb_r[...])
        if has_u:
            z = z + _dg(u_r[...], wu_r[...], 1, 0)
        uu = _relu(z)
        outs[0][...] = uu
        if has_proj:
            outs[1][...] = _dg(uu, a_r[...], 1, 0)
            outs[2][...] = _dg(uu, b_r[...], 1, 0)
        else:
            outs[1][...] = _dg(a_r[...], uu, 0, 1) + b_r[...]

    full = lambda a: pl.BlockSpec(a.shape, None)
    return pl.pallas_call(
        body,
        in_specs=[full(a) for a in in_arrays],
        out_specs=[pl.BlockSpec(s.shape, None) for s in out_shape],
        out_shape=out_shape,
    )(*in_arrays)


_NW = 32
_EPW = 25000
_CH = 1000
_NCH = _EPW // _CH
_NPAD = 50048
_NPT = _NPAD // 16


def _sc_mesh():
    return plsc.VectorSubcoreMesh(core_axis_name="c", subcore_axis_name="s")


def _compute_eb(src, batch):
    n = batch.shape[0]
    e_cnt = src.shape[0]

    @functools.partial(
        pl.kernel,
        mesh=_sc_mesh(),
        compiler_params=pltpu.CompilerParams(needs_layout_passes=False, use_tc_tiling_on_sc=False),
        out_type=jax.ShapeDtypeStruct((e_cnt,), jnp.int32),
        scratch_types=[
            pltpu.VMEM((n,), jnp.int32),
            pltpu.VMEM((_EPW,), jnp.int32),
            pltpu.VMEM((_EPW,), jnp.int32),
        ],
    )
    def k(src_hbm, batch_hbm, eb_out, batch_v, src_v, eb_v):
        wid = lax.axis_index("s") * 2 + lax.axis_index("c")
        base = wid * _EPW
        pltpu.sync_copy(batch_hbm, batch_v)
        pltpu.sync_copy(src_hbm.at[pl.ds(base, _EPW)], src_v)

        def body(j, _):
            off = jnp.minimum(j * 16, _EPW - 16)
            idx = src_v[pl.ds(off, 16)]
            eb_v[pl.ds(off, 16)] = plsc.load_gather(batch_v, [idx])
            return 0

        lax.fori_loop(0, (_EPW + 15) // 16, body, 0)
        pltpu.sync_copy(eb_v, eb_out.at[pl.ds(base, _EPW)])

    return k(src, batch)


def _edge_pass(p, q, src, eb):
    e_cnt = q.shape[0]

    @functools.partial(
        pl.kernel,
        mesh=_sc_mesh(),
        compiler_params=pltpu.CompilerParams(needs_layout_passes=False,
                                             use_tc_tiling_on_sc=False),
        out_type=[
            jax.ShapeDtypeStruct((e_cnt, 32), jnp.float32),
            jax.ShapeDtypeStruct((2, G, 32), jnp.float32),
        ],
        scratch_types=[
            pltpu.VMEM((_CH,), jnp.int32),
            pltpu.VMEM((_CH,), jnp.int32),
            pltpu.VMEM((_CH,), jnp.int32),
            pltpu.VMEM((_CH, 32), jnp.float32),
            pltpu.VMEM((_CH, 32), jnp.float32),
            pltpu.VMEM((_CH, 32), jnp.float32),
            pltpu.VMEM((16, 32), jnp.float32),
            pltpu.VMEM_SHARED((G, 32), jnp.float32),
            pltpu.SemaphoreType.DMA,
            pltpu.SemaphoreType.DMA,
            pltpu.SemaphoreType.DMA,
            pltpu.SemaphoreType.DMA,
        ],
    )
    def ka(p_hbm, q_hbm, src_hbm, eb_hbm, e_out, eg_out,
           src0, src1, eb_v, pg0, pg1, q_v, zb_v, eg_sh, sem0, sem1,
           sem_eb, sem_eo):
        c = lax.axis_index("c")
        s = lax.axis_index("s")
        wid = s * 2 + c
        zero16 = jnp.zeros((16,), jnp.float32)

        def zfill(i, _):
            zb_v[i, pl.ds(0, 16)] = zero16
            zb_v[i, pl.ds(16, 16)] = zero16
            return 0

        lax.fori_loop(0, 16, zfill, 0)
        pltpu.sync_copy(zb_v, eg_sh.at[pl.ds(s * 16, 16)])
        plsc.subcore_barrier()

        base = wid * _EPW
        sets = ((src0, pg0, sem0), (src1, pg1, sem1))

        def issue(g, st):
            src_v, pg_v, sem = st
            off = base + g * _CH
            pltpu.sync_copy(src_hbm.at[pl.ds(off, _CH)], src_v)
            pltpu.async_copy(p_hbm.at[src_v], pg_v, sem)

        def proc(g, st):
            src_v, pg_v, sem = st
            off = base + g * _CH
            pltpu.async_copy(eb_hbm.at[pl.ds(off, _CH)], eb_v, sem_eb)
            pltpu.make_async_copy(p_hbm.at[src_v], pg_v, sem).wait()
            pltpu.sync_copy(q_hbm.at[pl.ds(off, _CH)], q_v)

            def rows(rr, _):
                for u in range(4):
                    r = rr * 4 + u
                    a0 = pg_v[r, pl.ds(0, 16)] + q_v[r, pl.ds(0, 16)]
                    a1 = pg_v[r, pl.ds(16, 16)] + q_v[r, pl.ds(16, 16)]
                    q_v[r, pl.ds(0, 16)] = jnp.maximum(a0, 0.0)
                    q_v[r, pl.ds(16, 16)] = jnp.maximum(a1, 0.0)
                return 0

            lax.fori_loop(0, _CH // 4, rows, 0)
            pltpu.async_copy(q_v, e_out.at[pl.ds(off, _CH)], sem_eo)
            pltpu.make_async_copy(eb_hbm.at[pl.ds(off, _CH)], eb_v,
                                  sem_eb).wait()
            pltpu.sync_copy(q_v, eg_sh.at[eb_v], add=True)
            pltpu.make_async_copy(q_v, e_out.at[pl.ds(off, _CH)],
                                  sem_eo).wait()

        issue(0, sets[0])

        def pair(i, _):
            g = 2 * i
            issue(g + 1, sets[1])
            proc(g, sets[0])
            issue(g + 2, sets[0])
            proc(g + 1, sets[1])
            return 0

        lax.fori_loop(0, (_NCH - 1) // 2, pair, 0)
        proc(_NCH - 1, sets[0])

        plsc.subcore_barrier()
        pltpu.sync_copy(eg_sh.at[pl.ds(s * 16, 16)],
                        eg_out.at[c, pl.ds(s * 16, 16)])

    return ka(p, q, src, eb)


_QR = _NPAD // 4
_QPT = _QR // 16
_DUMP = 2048
_CHB = 1000
_EPT_B = 800000 // 16
_NCHB = _EPT_B // _CHB


def _msg_pass(dst, e_new):
    @functools.partial(
        pl.kernel,
        mesh=_sc_mesh(),
        compiler_params=pltpu.CompilerParams(needs_layout_passes=False,
                                             use_tc_tiling_on_sc=False),
        out_type=jax.ShapeDtypeStruct((_NPAD, 32), jnp.float32),
        scratch_types=[
            pltpu.VMEM((_CHB,), jnp.int32),
            pltpu.VMEM((_CHB,), jnp.int32),
            pltpu.VMEM((_CHB,), jnp.int32),
            pltpu.VMEM((_CHB,), jnp.int32),
            pltpu.VMEM((_CHB, 32), jnp.float32),
            pltpu.VMEM((_CHB, 32), jnp.float32),
            pltpu.VMEM((_QPT, 32), jnp.float32),
            pltpu.VMEM_SHARED((_QR + _DUMP, 32), jnp.float32),
            pltpu.SemaphoreType.DMA,
            pltpu.SemaphoreType.DMA,
            pltpu.SemaphoreType.DMA,
            pltpu.SemaphoreType.DMA,
        ],
    )
    def kb(dst_hbm, e_hbm, msg_out, d0, d1, iq0, iq1, r0, r1, zb_v, acc,
           semd0, seme0, semd1, seme1):
        c = lax.axis_index("c")
        s = lax.axis_index("s")
        zero16 = jnp.zeros((16,), jnp.float32)

        def zfill(i, _):
            zb_v[i, pl.ds(0, 16)] = zero16
            zb_v[i, pl.ds(16, 16)] = zero16
            return 0

        lax.fori_loop(0, _QPT, zfill, 0)

        sets = ((d0, iq0, r0, semd0, seme0), (d1, iq1, r1, semd1, seme1))

        def issue(g, st):
            d_v, iq_v, rows_v, semd, seme = st
            off = s * _EPT_B + g * _CHB
            pltpu.async_copy(dst_hbm.at[pl.ds(off, _CHB)], d_v, semd)
            pltpu.async_copy(e_hbm.at[pl.ds(off, _CHB)], rows_v, seme)

        for qq in range(2):
            qlo = (2 * c + qq) * _QR
            pltpu.sync_copy(zb_v, acc.at[pl.ds(s * _QPT, _QPT)])
            for t in range(_DUMP // _QPT + 1):
                lo = jnp.minimum(_QR + t * _QPT, _QR + _DUMP - _QPT)
                pltpu.sync_copy(zb_v, acc.at[pl.ds(lo, _QPT)])
            plsc.subcore_barrier()

            def proc(g, st):
                d_v, iq_v, rows_v, semd, seme = st
                off = s * _EPT_B + g * _CHB
                pltpu.make_async_copy(dst_hbm.at[pl.ds(off, _CHB)],
                                      d_v, semd).wait()

                def vec(ii, _):
                    for u in range(4):
                        i = ii * 4 + u
                        d = d_v[pl.ds(i * 16, 16)]
                        dq = d - qlo
                        m = (dq >= 0) & (dq < _QR)
                        sp = _QR + ((g * _CHB + i * 16
                                     + lax.broadcasted_iota(jnp.int32, (16,), 0))
                                    & (_DUMP - 1))
                        iq_v[pl.ds(i * 16, 16)] = jnp.where(m, dq, sp)
                    return 0

                lax.fori_loop(0, 15, vec, 0)
                for off_t in (960, 976, 984):
                    d = d_v[pl.ds(off_t, 16)]
                    dq = d - qlo
                    m = (dq >= 0) & (dq < _QR)
                    sp = _QR + ((g * _CHB + off_t
                                 + lax.broadcasted_iota(jnp.int32, (16,), 0))
                                & (_DUMP - 1))
                    iq_v[pl.ds(off_t, 16)] = jnp.where(m, dq, sp)
                pltpu.make_async_copy(e_hbm.at[pl.ds(off, _CHB)],
                                      rows_v, seme).wait()
                pltpu.sync_copy(rows_v, acc.at[iq_v], add=True)

            issue(0, sets[0])

            def pair(i, _):
                g = 2 * i
                issue(g + 1, sets[1])
                proc(g, sets[0])
                issue(g + 2, sets[0])
                proc(g + 1, sets[1])
                return 0

            lax.fori_loop(0, (_NCHB - 2) // 2, pair, 0)
            issue(_NCHB - 1, sets[1])
            proc(_NCHB - 2, sets[0])
            proc(_NCHB - 1, sets[1])

            plsc.subcore_barrier()
            pltpu.sync_copy(acc.at[pl.ds(s * _QPT, _QPT)],
                            msg_out.at[pl.ds(qlo + s * _QPT, _QPT)])
            plsc.subcore_barrier()

    return kb(dst, e_new)


def kernel(x, edge_index, edge_attr, batch,
           en_W1, en_b1, en_W2, en_b2,
           ee_W1, ee_b1, ee_W2, ee_b2,
           l1_eW, l1_eb, l1_nW, l1_nb, l1_gW, l1_gb,
           l2_eW, l2_eb, l2_nW, l2_nb, l2_gW, l2_gb,
           l3_eW, l3_eb, l3_nW, l3_nb, l3_gW, l3_gb,
           ro_W, ro_b):
    n = x.shape[0]
    e_cnt = edge_index.shape[1]
    src = edge_index[0]
    dst = edge_index[1]
    batch3 = batch.reshape(n // BN, 1, BN)

    row = lambda v: v.reshape(1, -1)
    col = lambda v: v.reshape(-1, 1)

    x0, p1 = _enc_node(x, en_W1, row(en_b1), en_W2, row(en_b2), l1_eW[:128])
    q1 = _enc_edge_q1(edge_attr.reshape(1, e_cnt), col(ee_W1[0]),
                      col(ee_b1), ee_W2.T, col(ee_b2),
                      l1_eW[128:160], row(l1_eb))

    eb_arr = _compute_eb(src, batch)

    layers = [
        (l1_eW, l1_eb, l1_nW, l1_nb, l1_gW, l1_gb, False),
        (l2_eW, l2_eb, l2_nW, l2_nb, l2_gW, l2_gb, True),
        (l3_eW, l3_eb, l3_nW, l3_nb, l3_gW, l3_gb, True),
    ]

    xc = x0
    u = None
    uproj_e = None
    uproj_n = None
    q = q1
    e_arr = None
    res = None

    p_part = p1
    for li, (eW, ebias, nW, nb, gW, gb, has_u) in enumerate(layers):
        if li > 0:
            q = _q_mat(e_arr, eW[128:160], row(ebias))
        p = _p_add(p_part, batch3, uproj_e) if has_u else p_part

        e_arr, eg_p = _edge_pass(p, q, src, eb_arr)
        msg = _msg_pass(dst, e_arr)

        is_last = li == 2
        wnext = None if is_last else layers[li + 1][0][:128]
        outs_n = _node_mat(xc, msg, batch3, nW[:128], nW[128:160],
                           row(nb), uproj_n if has_u else None, wnext)
        xc, xg = outs_n[0], outs_n[1]
        if not is_last:
            p_part = outs_n[2]

        if not is_last:
            nxt_eW = layers[li + 1][0]
            nxt_nW = layers[li + 1][2]
            outs = _global_mat(eg_p, xg, u, gW[:32], gW[32:160],
                               gW[160:192] if has_u else None, row(gb),
                               eproj_w=nxt_eW[160:192], nproj_w=nxt_nW[160:192])
            u, uproj_e, uproj_n = outs
        else:
            outs = _global_mat(eg_p, xg, u, gW[:32], gW[32:160],
                               gW[160:192], row(gb),
                               ro_w=ro_W, ro_br=row(ro_b))
            res = outs[1]

    return res.reshape(-1)


# speedup vs baseline: 5.8394x; 1.1783x over previous
import functools

import jax
import jax.numpy as jnp
from jax import lax
from jax.experimental import pallas as pl
from jax.experimental.pallas import tpu as pltpu
from jax.experimental.pallas import tpu_sc as plsc

G = 256

BN = 2000
BE = 8000
BA = 6400


_PREC = lax.Precision.HIGHEST
_PREC_D = lax.Precision.DEFAULT


def _dg(a, b, ca, cb, prec=_PREC_D):
    return lax.dot_general(a, b, (((ca,), (cb,)), ((), ())),
                           preferred_element_type=jnp.float32,
                           precision=prec)


def _relu(v):
    return jnp.maximum(v, 0.0)


def _enc_node_body(x_ref, w1, b1, w2, b2, wx1, o_ref, p_ref):
    h = _relu(jnp.dot(x_ref[...], w1[...], precision=_PREC_D,
                      preferred_element_type=jnp.float32) + b1[...])
    x0 = _relu(jnp.dot(h, w2[...], precision=_PREC_D,
                       preferred_element_type=jnp.float32) + b2[...])
    o_ref[...] = x0
    p_ref[...] = jnp.dot(x0, wx1[...], precision=_PREC_D,
                         preferred_element_type=jnp.float32)


def _enc_node(x, w1, b1r, w2, b2r, wx1):
    n = x.shape[0]
    grid = n // BN
    return pl.pallas_call(
        _enc_node_body,
        grid=(grid,),
        in_specs=[
            pl.BlockSpec((BN, x.shape[1]), lambda i: (i, 0)),
            pl.BlockSpec(w1.shape, lambda i: (0, 0)),
            pl.BlockSpec(b1r.shape, lambda i: (0, 0)),
            pl.BlockSpec(w2.shape, lambda i: (0, 0)),
            pl.BlockSpec(b2r.shape, lambda i: (0, 0)),
            pl.BlockSpec(wx1.shape, lambda i: (0, 0)),
        ],
        out_specs=[
            pl.BlockSpec((BN, w2.shape[1]), lambda i: (i, 0)),
            pl.BlockSpec((BN, 32), lambda i: (i, 0)),
        ],
        out_shape=[
            jax.ShapeDtypeStruct((n, w2.shape[1]), jnp.float32),
            jax.ShapeDtypeStruct((n, 32), jnp.float32),
        ],
    )(x, w1, b1r, w2, b2r, wx1)


def _enc_edge_body(a_ref, w1c, b1c, w2t, b2c, we, ber, o_ref):
    bf = lambda v: v.astype(jnp.bfloat16).astype(jnp.float32)
    a = bf(a_ref[...])
    h1 = _relu(bf(w1c[...]) * a + b1c[...])
    e0 = _relu(_dg(w2t[...], h1, 1, 0) + b2c[...])
    o_ref[...] = _dg(e0, we[...], 0, 0) + ber[...]


def _enc_edge_q1(a_row, w1c, b1c, w2t, b2c, we, ber):
    e = a_row.shape[1]
    grid = e // BA
    return pl.pallas_call(
        _enc_edge_body,
        grid=(grid,),
        in_specs=[
            pl.BlockSpec((1, BA), lambda i: (0, i)),
            pl.BlockSpec(w1c.shape, lambda i: (0, 0)),
            pl.BlockSpec(b1c.shape, lambda i: (0, 0)),
            pl.BlockSpec(w2t.shape, lambda i: (0, 0)),
            pl.BlockSpec(b2c.shape, lambda i: (0, 0)),
            pl.BlockSpec(we.shape, lambda i: (0, 0)),
            pl.BlockSpec(ber.shape, lambda i: (0, 0)),
        ],
        out_specs=pl.BlockSpec((BA, 32), lambda i: (i, 0)),
        out_shape=jax.ShapeDtypeStruct((e, 32), jnp.float32),
    )(a_row, w1c, b1c, w2t, b2c, we, ber)


def _q_body(e_ref, we, ber, o_ref):
    o_ref[...] = jnp.dot(e_ref[...], we[...], precision=_PREC_D,
                         preferred_element_type=jnp.float32) + ber[...]


def _q_mat(e, we, ber):
    n = e.shape[0]
    grid = n // BE
    return pl.pallas_call(
        _q_body,
        grid=(grid,),
        in_specs=[
            pl.BlockSpec((BE, 32), lambda i: (i, 0)),
            pl.BlockSpec(we.shape, lambda i: (0, 0)),
            pl.BlockSpec(ber.shape, lambda i: (0, 0)),
        ],
        out_specs=pl.BlockSpec((BE, 32), lambda i: (i, 0)),
        out_shape=jax.ShapeDtypeStruct((n, 32), jnp.float32),
    )(e, we, ber)


def _onehot_t(batch_row):
    segs = lax.broadcasted_iota(jnp.int32, (G, batch_row.shape[1]), 0)
    return jnp.where(segs == batch_row, 1.0, 0.0).astype(jnp.float32)


def _padd_body(p_ref, b3_ref, uproj, o_ref):
    oh = _onehot_t(b3_ref[0])
    o_ref[...] = p_ref[...] + _dg(oh, uproj[...], 0, 0, _PREC)


def _p_add(p_part, batch3, uproj):
    n = p_part.shape[0]
    grid = n // BN
    return pl.pallas_call(
        _padd_body,
        grid=(grid,),
        in_specs=[
            pl.BlockSpec((BN, 32), lambda i: (i, 0)),
            pl.BlockSpec((1, 1, BN), lambda i: (i, 0, 0)),
            pl.BlockSpec(uproj.shape, lambda i: (0, 0)),
        ],
        out_specs=pl.BlockSpec((BN, 32), lambda i: (i, 0)),
        out_shape=jax.ShapeDtypeStruct((n, 32), jnp.float32),
    )(p_part, batch3, uproj)


def _node_body(x_ref, m_ref, b3_ref, wx, wm, nbr, x_out, xg_out, *, has_u,
               uproj_ref=None, wnext_ref=None, p_out=None):
    m = m_ref[...]
    z = (jnp.dot(x_ref[...], wx[...], precision=_PREC_D, preferred_element_type=jnp.float32)
         + _dg(m, wm[...], 1, 0) + nbr[...])
    oh = _onehot_t(b3_ref[0])
    if has_u:
        z = z + _dg(oh, uproj_ref[...], 0, 0, _PREC)
    xn = _relu(z)
    x_out[...] = xn
    if p_out is not None:
        p_out[...] = jnp.dot(xn, wnext_ref[...], precision=_PREC_D,
                             preferred_element_type=jnp.float32)

    @pl.when(pl.program_id(0) == 0)
    def _():
        xg_out[...] = jnp.zeros_like(xg_out)

    xg_out[...] += _dg(oh, xn, 1, 0, _PREC)


def _node_mat(x, msg_p, batch3, wx, wm, nbr, uprojn, wnext):
    n = x.shape[0]
    grid = n // BN
    has_u = uprojn is not None
    has_p = wnext is not None

    extra_specs = []
    args = [x, msg_p, batch3, wx, wm, nbr]
    if has_u:
        extra_specs.append(pl.BlockSpec(uprojn.shape, lambda i: (0, 0)))
        args.append(uprojn)
    if has_p:
        extra_specs.append(pl.BlockSpec(wnext.shape, lambda i: (0, 0)))
        args.append(wnext)

    def body(*refs):
        k = 6
        x_ref, m_ref, b3_ref, wx_r, wm_r, nb_r = refs[:k]
        up_r = wn_r = None
        if has_u:
            up_r = refs[k]; k += 1
        if has_p:
            wn_r = refs[k]; k += 1
        outs = refs[k:]
        p_o = outs[2] if has_p else None
        _node_body(x_ref, m_ref, b3_ref, wx_r, wm_r, nb_r, outs[0], outs[1],
                   has_u=has_u, uproj_ref=up_r, wnext_ref=wn_r, p_out=p_o)

    out_specs = [
        pl.BlockSpec((BN, 128), lambda i: (i, 0)),
        pl.BlockSpec((G, 128), lambda i: (0, 0)),
    ]
    out_shape = [
        jax.ShapeDtypeStruct((n, 128), jnp.float32),
        jax.ShapeDtypeStruct((G, 128), jnp.float32),
    ]
    if has_p:
        out_specs.append(pl.BlockSpec((BN, 32), lambda i: (i, 0)))
        out_shape.append(jax.ShapeDtypeStruct((n, 32), jnp.float32))

    return pl.pallas_call(
        body,
        grid=(grid,),
        in_specs=[
            pl.BlockSpec((BN, 128), lambda i: (i, 0)),
            pl.BlockSpec((BN, 32), lambda i: (i, 0)),
            pl.BlockSpec((1, 1, BN), lambda i: (i, 0, 0)),
            pl.BlockSpec(wx.shape, lambda i: (0, 0)),
            pl.BlockSpec(wm.shape, lambda i: (0, 0)),
            pl.BlockSpec(nbr.shape, lambda i: (0, 0)),
        ] + extra_specs,
        out_specs=out_specs,
        out_shape=out_shape,
    )(*args)


def _global_mat(eg_p, xg, u_prev, we, wxg, wu, gbr,
                eproj_w=None, nproj_w=None, ro_w=None, ro_br=None):
    has_u = u_prev is not None
    has_proj = eproj_w is not None

    in_arrays = [eg_p, xg, we, wxg, gbr]
    if has_u:
        in_arrays += [u_prev, wu]
    out_shape = [jax.ShapeDtypeStruct((G, 32), jnp.float32)]
    if has_proj:
        in_arrays += [eproj_w, nproj_w]
        out_shape += [jax.ShapeDtypeStruct((G, 32), jnp.float32),
                      jax.ShapeDtypeStruct((G, 128), jnp.float32)]
    else:
        in_arrays += [ro_w, ro_br]
        out_shape += [jax.ShapeDtypeStruct((1, G), jnp.float32)]

    def body(*refs):
        k = 5
        egp_r, xg_r, we_r, wxg_r, gb_r = refs[:k]
        if has_u:
            u_r, wu_r = refs[k:k + 2]
            k += 2
        a_r, b_r = refs[k:k + 2]
        k += 2
        outs = refs[k:]
        eg = egp_r[0] + egp_r[1]
        z = (_dg(eg, we_r[...], 1, 0) + _dg(xg_r[...], wxg_r[...], 1, 0)
             + gb_r[...])
        if has_u:
            z = z + _dg(u_r[...], wu_r[...], 1, 0)
        uu = _relu(z)
        outs[0][...] = uu
        if has_proj:
            outs[1][...] = _dg(uu, a_r[...], 1, 0)
            outs[2][...] = _dg(uu, b_r[...], 1, 0)
        else:
            outs[1][...] = _dg(a_r[...], uu, 0, 1) + b_r[...]

    full = lambda a: pl.BlockSpec(a.shape, None)
    return pl.pallas_call(
        body,
        in_specs=[full(a) for a in in_arrays],
        out_specs=[pl.BlockSpec(s.shape, None) for s in out_shape],
        out_shape=out_shape,
    )(*in_arrays)


_NW = 32
_EPW = 25000
_CH = 1000
_NCH = _EPW // _CH
_NPAD = 50048
_NPT = _NPAD // 16


def _sc_mesh():
    return plsc.VectorSubcoreMesh(core_axis_name="c", subcore_axis_name="s")


def _compute_eb(src, batch):
    n = batch.shape[0]
    e_cnt = src.shape[0]

    @functools.partial(
        pl.kernel,
        mesh=_sc_mesh(),
        compiler_params=pltpu.CompilerParams(needs_layout_passes=False, use_tc_tiling_on_sc=False),
        out_type=jax.ShapeDtypeStruct((e_cnt,), jnp.int32),
        scratch_types=[
            pltpu.VMEM((n,), jnp.int32),
            pltpu.VMEM((_EPW,), jnp.int32),
            pltpu.VMEM((_EPW,), jnp.int32),
        ],
    )
    def k(src_hbm, batch_hbm, eb_out, batch_v, src_v, eb_v):
        wid = lax.axis_index("s") * 2 + lax.axis_index("c")
        base = wid * _EPW
        pltpu.sync_copy(batch_hbm, batch_v)
        pltpu.sync_copy(src_hbm.at[pl.ds(base, _EPW)], src_v)

        def body(j, _):
            off = jnp.minimum(j * 16, _EPW - 16)
            idx = src_v[pl.ds(off, 16)]
            eb_v[pl.ds(off, 16)] = plsc.load_gather(batch_v, [idx])
            return 0

        lax.fori_loop(0, (_EPW + 15) // 16, body, 0)
        pltpu.sync_copy(eb_v, eb_out.at[pl.ds(base, _EPW)])

    return k(src, batch)


def _edge_pass(p, q, src, eb):
    e_cnt = q.shape[0]

    @functools.partial(
        pl.kernel,
        mesh=_sc_mesh(),
        compiler_params=pltpu.CompilerParams(needs_layout_passes=False,
                                             use_tc_tiling_on_sc=False),
        out_type=[
            jax.ShapeDtypeStruct((e_cnt, 32), jnp.float32),
            jax.ShapeDtypeStruct((2, G, 32), jnp.float32),
        ],
        scratch_types=[
            pltpu.VMEM((_CH,), jnp.int32),
            pltpu.VMEM((_CH,), jnp.int32),
            pltpu.VMEM((_CH,), jnp.int32),
            pltpu.VMEM((_CH, 32), jnp.float32),
            pltpu.VMEM((_CH, 32), jnp.float32),
            pltpu.VMEM((_CH, 32), jnp.float32),
            pltpu.VMEM((16, 32), jnp.float32),
            pltpu.VMEM_SHARED((G, 32), jnp.float32),
            pltpu.SemaphoreType.DMA,
            pltpu.SemaphoreType.DMA,
            pltpu.SemaphoreType.DMA,
            pltpu.SemaphoreType.DMA,
        ],
    )
    def ka(p_hbm, q_hbm, src_hbm, eb_hbm, e_out, eg_out,
           src0, src1, eb_v, pg0, pg1, q_v, zb_v, eg_sh, sem0, sem1,
           sem_eb, sem_eo):
        c = lax.axis_index("c")
        s = lax.axis_index("s")
        wid = s * 2 + c
        zero16 = jnp.zeros((16,), jnp.float32)

        def zfill(i, _):
            zb_v[i, pl.ds(0, 16)] = zero16
            zb_v[i, pl.ds(16, 16)] = zero16
            return 0

        lax.fori_loop(0, 16, zfill, 0)
        pltpu.sync_copy(zb_v, eg_sh.at[pl.ds(s * 16, 16)])
        plsc.subcore_barrier()

        base = wid * _EPW
        sets = ((src0, pg0, sem0), (src1, pg1, sem1))

        def issue(g, st):
            src_v, pg_v, sem = st
            off = base + g * _CH
            pltpu.sync_copy(src_hbm.at[pl.ds(off, _CH)], src_v)
            pltpu.async_copy(p_hbm.at[src_v], pg_v, sem)

        def proc(g, st):
            src_v, pg_v, sem = st
            off = base + g * _CH
            pltpu.async_copy(eb_hbm.at[pl.ds(off, _CH)], eb_v, sem_eb)
            pltpu.make_async_copy(p_hbm.at[src_v], pg_v, sem).wait()
            pltpu.sync_copy(q_hbm.at[pl.ds(off, _CH)], q_v)

            def rows(rr, _):
                for u in range(4):
                    r = rr * 4 + u
                    a0 = pg_v[r, pl.ds(0, 16)] + q_v[r, pl.ds(0, 16)]
                    a1 = pg_v[r, pl.ds(16, 16)] + q_v[r, pl.ds(16, 16)]
                    q_v[r, pl.ds(0, 16)] = jnp.maximum(a0, 0.0)
                    q_v[r, pl.ds(16, 16)] = jnp.maximum(a1, 0.0)
                return 0

            lax.fori_loop(0, _CH // 4, rows, 0)
            pltpu.async_copy(q_v, e_out.at[pl.ds(off, _CH)], sem_eo)
            pltpu.make_async_copy(eb_hbm.at[pl.ds(off, _CH)], eb_v,
                                  sem_eb).wait()
            pltpu.sync_copy(q_v, eg_sh.at[eb_v], add=True)
            pltpu.make_async_copy(q_v, e_out.at[pl.ds(off, _CH)],
                                  sem_eo).wait()

        issue(0, sets[0])

        def pair(i, _):
            g = 2 * i
            issue(g + 1, sets[1])
            proc(g, sets[0])
            issue(g + 2, sets[0])
            proc(g + 1, sets[1])
            return 0

        lax.fori_loop(0, (_NCH - 1) // 2, pair, 0)
        proc(_NCH - 1, sets[0])

        plsc.subcore_barrier()
        pltpu.sync_copy(eg_sh.at[pl.ds(s * 16, 16)],
                        eg_out.at[c, pl.ds(s * 16, 16)])

    return ka(p, q, src, eb)


_QR = _NPAD // 4
_QPT = _QR // 16
_DUMP = 2048
_CHB = 1000
_EPT_B = 800000 // 16
_NCHB = _EPT_B // _CHB


def _msg_pass(dst, e_new):
    @functools.partial(
        pl.kernel,
        mesh=_sc_mesh(),
        compiler_params=pltpu.CompilerParams(needs_layout_passes=False,
                                             use_tc_tiling_on_sc=False),
        out_type=jax.ShapeDtypeStruct((_NPAD, 32), jnp.float32),
        scratch_types=[
            pltpu.VMEM((_CHB,), jnp.int32),
            pltpu.VMEM((_CHB,), jnp.int32),
            pltpu.VMEM((_CHB,), jnp.int32),
            pltpu.VMEM((_CHB,), jnp.int32),
            pltpu.VMEM((_CHB, 32), jnp.float32),
            pltpu.VMEM((_CHB, 32), jnp.float32),
            pltpu.VMEM((_QPT, 32), jnp.float32),
            pltpu.VMEM_SHARED((_QR + _DUMP, 32), jnp.float32),
            pltpu.SemaphoreType.DMA,
            pltpu.SemaphoreType.DMA,
            pltpu.SemaphoreType.DMA,
            pltpu.SemaphoreType.DMA,
        ],
    )
    def kb(dst_hbm, e_hbm, msg_out, d0, d1, iq0, iq1, r0, r1, zb_v, acc,
           semd0, seme0, semd1, seme1):
        c = lax.axis_index("c")
        s = lax.axis_index("s")
        zero16 = jnp.zeros((16,), jnp.float32)

        def zfill(i, _):
            zb_v[i, pl.ds(0, 16)] = zero16
            zb_v[i, pl.ds(16, 16)] = zero16
            return 0

        lax.fori_loop(0, _QPT, zfill, 0)

        sets = ((d0, iq0, r0, semd0, seme0), (d1, iq1, r1, semd1, seme1))

        def issue(g, st):
            d_v, iq_v, rows_v, semd, seme = st
            off = s * _EPT_B + g * _CHB
            pltpu.async_copy(dst_hbm.at[pl.ds(off, _CHB)], d_v, semd)
            pltpu.async_copy(e_hbm.at[pl.ds(off, _CHB)], rows_v, seme)

        for qq in range(2):
            qlo = (2 * c + qq) * _QR
            pltpu.sync_copy(zb_v, acc.at[pl.ds(s * _QPT, _QPT)])
            for t in range(_DUMP // _QPT + 1):
                lo = jnp.minimum(_QR + t * _QPT, _QR + _DUMP - _QPT)
                pltpu.sync_copy(zb_v, acc.at[pl.ds(lo, _QPT)])
            plsc.subcore_barrier()

            def proc(g, st):
                d_v, iq_v, rows_v, semd, seme = st
                off = s * _EPT_B + g * _CHB
                pltpu.make_async_copy(dst_hbm.at[pl.ds(off, _CHB)],
                                      d_v, semd).wait()

                def vec(ii, _):
                    for u in range(4):
                        i = ii * 4 + u
                        d = d_v[pl.ds(i * 16, 16)]
                        dq = d - qlo
                        m = (dq >= 0) & (dq < _QR)
                        sp = _QR + ((g * _CHB + i * 16
                                     + lax.broadcasted_iota(jnp.int32, (16,), 0))
                                    & (_DUMP - 1))
                        iq_v[pl.ds(i * 16, 16)] = jnp.where(m, dq, sp)
                    return 0

                lax.fori_loop(0, 15, vec, 0)
                for off_t in (960, 976, 984):
                    d = d_v[pl.ds(off_t, 16)]
                    dq = d - qlo
                    m = (dq >= 0) & (dq < _QR)
                    sp = _QR + ((g * _CHB + off_t
                                 + lax.broadcasted_iota(jnp.int32, (16,), 0))
                                & (_DUMP - 1))
                    iq_v[pl.ds(off_t, 16)] = jnp.where(m, dq, sp)
                pltpu.make_async_copy(e_hbm.at[pl.ds(off, _CHB)],
                                      rows_v, seme).wait()
                pltpu.sync_copy(rows_v, acc.at[iq_v], add=True)

            issue(0, sets[0])

            def pair(i, _):
                g = 2 * i
                issue(g + 1, sets[1])
                proc(g, sets[0])
                issue(g + 2, sets[0])
                proc(g + 1, sets[1])
                return 0

            lax.fori_loop(0, (_NCHB - 2) // 2, pair, 0)
            issue(_NCHB - 1, sets[1])
            proc(_NCHB - 2, sets[0])
            proc(_NCHB - 1, sets[1])

            plsc.subcore_barrier()
            pltpu.sync_copy(acc.at[pl.ds(s * _QPT, _QPT)],
                            msg_out.at[pl.ds(qlo + s * _QPT, _QPT)])
            plsc.subcore_barrier()

    return kb(dst, e_new)


def kernel(x, edge_index, edge_attr, batch,
           en_W1, en_b1, en_W2, en_b2,
           ee_W1, ee_b1, ee_W2, ee_b2,
           l1_eW, l1_eb, l1_nW, l1_nb, l1_gW, l1_gb,
           l2_eW, l2_eb, l2_nW, l2_nb, l2_gW, l2_gb,
           l3_eW, l3_eb, l3_nW, l3_nb, l3_gW, l3_gb,
           ro_W, ro_b):
    n = x.shape[0]
    e_cnt = edge_index.shape[1]
    src = edge_index[0]
    dst = edge_index[1]
    batch3 = batch.reshape(n // BN, 1, BN)

    row = lambda v: v.reshape(1, -1)
    col = lambda v: v.reshape(-1, 1)

    x0, p1 = _enc_node(x, en_W1, row(en_b1), en_W2, row(en_b2), l1_eW[:128])
    q1 = _enc_edge_q1(edge_attr.reshape(1, e_cnt), col(ee_W1[0]),
                      col(ee_b1), ee_W2.T, col(ee_b2),
                      l1_eW[128:160], row(l1_eb))

    eb_arr = _compute_eb(src, batch)

    layers = [
        (l1_eW, l1_eb, l1_nW, l1_nb, l1_gW, l1_gb, False),
        (l2_eW, l2_eb, l2_nW, l2_nb, l2_gW, l2_gb, True),
        (l3_eW, l3_eb, l3_nW, l3_nb, l3_gW, l3_gb, True),
    ]

    xc = x0
    u = None
    uproj_e = None
    uproj_n = None
    q = q1
    e_arr = None
    res = None

    p_part = p1
    for li, (eW, ebias, nW, nb, gW, gb, has_u) in enumerate(layers):
        if li > 0:
            q = _q_mat(e_arr, eW[128:160], row(ebias))
        p = _p_add(p_part, batch3, uproj_e) if has_u else p_part

        e_arr, eg_p = _edge_pass(p, q, src, eb_arr)
        msg = _msg_pass(dst, e_arr)

        is_last = li == 2
        wnext = None if is_last else layers[li + 1][0][:128]
        outs_n = _node_mat(xc, msg, batch3, nW[:128], nW[128:160],
                           row(nb), uproj_n if has_u else None, wnext)
        xc, xg = outs_n[0], outs_n[1]
        if not is_last:
            p_part = outs_n[2]

        if not is_last:
            nxt_eW = layers[li + 1][0]
            nxt_nW = layers[li + 1][2]
            outs = _global_mat(eg_p, xg, u, gW[:32], gW[32:160],
                               gW[160:192] if has_u else None, row(gb),
                               eproj_w=nxt_eW[160:192], nproj_w=nxt_nW[160:192])
            u, uproj_e, uproj_n = outs
        else:
            outs = _global_mat(eg_p, xg, u, gW[:32], gW[32:160],
                               gW[160:192], row(gb),
                               ro_w=ro_W, ro_br=row(ro_b))
            res = outs[1]

    return res.reshape(-1)
